# baseline (speedup 1.0000x reference)
import jax
import jax.numpy as jnp
from jax import lax
from jax.experimental import pallas as pl
from jax.experimental.pallas import tpu as pltpu
from jax.experimental.pallas import tpu_sc as plsc

NU = 10000
NI = 10000
E = 320000
EMB = 128
HID = 128

NPAD = 10240
RPT = NPAD // 16
K = 128
CHUNKS = 160
EPT = CHUNKS * K
EPAD = EPT * 16
NBUF = 2
NBLK = 40
BLK = NPAD // NBLK
GPW = NPAD // 32
GK = 80


def _embed_hist_body(ids_ref, utab_ref, itab_ref, dst_ref,
                     z_ref, cnt_ref,
                     idx_v, gidx_v, grow_v, ones_v, zrow_v, cacc_sh, sem):
    c = lax.axis_index("c")
    s = lax.axis_index("s")
    wid = c * 16 + s
    zeros16 = jnp.zeros((16,), jnp.float32)
    ones16 = jnp.ones((16,), jnp.float32)

    def obody(i, _):
        ones_v[pl.ds(i * 16, 16)] = ones16
        return 0
    lax.fori_loop(0, K // 16, obody, 0)

    def zcbody(i, _):
        zrow_v[pl.ds(i * 16, 16)] = zeros16
        return 0
    lax.fori_loop(0, RPT // 16, zcbody, 0)
    pltpu.sync_copy(zrow_v, cacc_sh.at[pl.ds(s * RPT, RPT)])
    plsc.subcore_barrier()

    for j in range(GPW // GK):
        base = wid * GPW + j * GK
        pltpu.sync_copy(ids_ref.at[pl.ds(base, GK)], gidx_v)
        pltpu.async_copy(utab_ref.at[gidx_v], grow_v, sem).wait()
        pltpu.sync_copy(grow_v, z_ref.at[pl.ds(base, GK)])
    for j in range(GPW // GK):
        base = NPAD + wid * GPW + j * GK
        pltpu.sync_copy(ids_ref.at[pl.ds(base, GK)], gidx_v)
        pltpu.async_copy(itab_ref.at[gidx_v], grow_v, sem).wait()
        pltpu.sync_copy(grow_v, z_ref.at[pl.ds(base, GK)])

    def hbody(j, _):
        off = c * EPAD + s * EPT + j * K
        pltpu.sync_copy(dst_ref.at[pl.ds(off, K)], idx_v)
        pltpu.sync_copy(ones_v, cacc_sh.at[idx_v], add=True)
        return 0
    lax.fori_loop(0, CHUNKS, hbody, 0)

    plsc.subcore_barrier()
    pltpu.sync_copy(cacc_sh.at[pl.ds(s * RPT, RPT)],
                    cnt_ref.at[pl.ds(c * NPAD + s * RPT, RPT)])


def _segsum_body(t_ref, src_ref, dst_ref, s_out_ref,
                 isrc_v, idst_v, rows_v, acc_sh, sem):
    c = lax.axis_index("c")
    s = lax.axis_index("s")
    ebase = c * EPAD + s * EPT
    zeros16 = jnp.zeros((16,), jnp.float32)

    def zbody(i, _):
        for k in range(EMB // 16):
            rows_v[i, pl.ds(k * 16, 16)] = zeros16
        return 0
    lax.fori_loop(0, K, zbody, 0)
    for j in range(RPT // K):
        pltpu.sync_copy(rows_v, acc_sh.at[pl.ds(s * RPT + j * K, K)])
    plsc.subcore_barrier()

    def body(j, _):
        off = ebase + j * K
        pltpu.sync_copy(src_ref.at[pl.ds(off, K)], isrc_v)
        pltpu.sync_copy(dst_ref.at[pl.ds(off, K)], idst_v)
        pltpu.async_copy(t_ref.at[isrc_v], rows_v, sem).wait()
        pltpu.sync_copy(rows_v, acc_sh.at[idst_v], add=True)
        return 0
    lax.fori_loop(0, CHUNKS, body, 0)

    plsc.subcore_barrier()
    pltpu.sync_copy(acc_sh.at[pl.ds(s * RPT, RPT)],
                    s_out_ref.at[pl.ds(c * NPAD + s * RPT, RPT)])


def _dense_layer_body(s_ref, c_ref, z_ref,
                      wnu_ref, wsu_ref, bu_ref, wni_ref, wsi_ref, bi_ref,
                      h_ref):
    cu = jnp.clip(c_ref[1, 0, 0, :], 1.0, None)
    ci = jnp.clip(c_ref[0, 0, 0, :], 1.0, None)
    agg_u = s_ref[1] / cu[:, None]
    agg_i = s_ref[0] / ci[:, None]
    nu = (jnp.dot(agg_u, wnu_ref[...], preferred_element_type=jnp.float32)
          + jnp.dot(z_ref[0], wsu_ref[...], preferred_element_type=jnp.float32)
          + bu_ref[0, :])
    ni = (jnp.dot(agg_i, wni_ref[...], preferred_element_type=jnp.float32)
          + jnp.dot(z_ref[1], wsi_ref[...], preferred_element_type=jnp.float32)
          + bi_ref[0, :])
    h_ref[0] = jnp.maximum(nu, 0.0)
    h_ref[1] = jnp.maximum(ni, 0.0)


def _dense_final_body(s_ref, c_ref, z_ref,
                      wnu_ref, wsu_ref, bu_ref, wni_ref, wsi_ref, bi_ref,
                      wp_ref, bp_ref,
                      out_ref):
    cu = jnp.clip(c_ref[1, 0, 0, :], 1.0, None)
    ci = jnp.clip(c_ref[0, 0, 0, :], 1.0, None)
    agg_u = s_ref[1] / cu[:, None]
    agg_i = s_ref[0] / ci[:, None]
    nu = (jnp.dot(agg_u, wnu_ref[...], preferred_element_type=jnp.float32)
          + jnp.dot(z_ref[0], wsu_ref[...], preferred_element_type=jnp.float32)
          + bu_ref[0, :])
    ni = (jnp.dot(agg_i, wni_ref[...], preferred_element_type=jnp.float32)
          + jnp.dot(z_ref[1], wsi_ref[...], preferred_element_type=jnp.float32)
          + bi_ref[0, :])
    hu = jnp.maximum(nu, 0.0)
    hi = jnp.maximum(ni, 0.0)
    res = (jnp.sum(hu * wp_ref[0, :][None, :], axis=1)
           + jnp.sum(hi * wp_ref[1, :][None, :], axis=1)
           + bp_ref[0, 0])
    out_ref[0, 0, :] = res


def _sc_mesh():
    return plsc.VectorSubcoreMesh(core_axis_name="c", subcore_axis_name="s")


def _embed_hist(ids_all, user_table, item_table, dst_all):
    return pl.kernel(
        _embed_hist_body,
        out_type=(
            jax.ShapeDtypeStruct((2 * NPAD, EMB), jnp.float32),
            jax.ShapeDtypeStruct((2 * NPAD,), jnp.float32),
        ),
        mesh=_sc_mesh(),
        scratch_types=[
            pltpu.VMEM((K,), jnp.int32),
            pltpu.VMEM((GK,), jnp.int32),
            pltpu.VMEM((GK, EMB), jnp.float32),
            pltpu.VMEM((K,), jnp.float32),
            pltpu.VMEM((RPT,), jnp.float32),
            pltpu.VMEM_SHARED((NPAD,), jnp.float32),
            pltpu.SemaphoreType.DMA,
        ],
    )(ids_all, user_table, item_table, dst_all)


def _segsum(table_flat, src_all, dst_all):
    return pl.kernel(
        _segsum_body,
        out_type=jax.ShapeDtypeStruct((2 * NPAD, EMB), jnp.float32),
        mesh=_sc_mesh(),
        scratch_types=[
            pltpu.VMEM((K,), jnp.int32),
            pltpu.VMEM((K,), jnp.int32),
            pltpu.VMEM((K, EMB), jnp.float32),
            pltpu.VMEM_SHARED((NPAD, EMB), jnp.float32),
            pltpu.SemaphoreType.DMA,
        ],
    )(table_flat, src_all, dst_all)


def _dense_layer(S, C4, Z, Wnu, Wsu, bu, Wni, Wsi, bi):
    wspec = pl.BlockSpec((EMB, HID), lambda i: (0, 0))
    bspec = pl.BlockSpec((1, HID), lambda i: (0, 0))
    return pl.pallas_call(
        _dense_layer_body,
        grid=(NBLK,),
        in_specs=[
            pl.BlockSpec((2, BLK, EMB), lambda i: (0, i, 0)),
            pl.BlockSpec((2, 1, 1, BLK), lambda i: (0, i, 0, 0)),
            pl.BlockSpec((2, BLK, EMB), lambda i: (0, i, 0)),
            wspec, wspec, bspec, wspec, wspec, bspec,
        ],
        out_specs=pl.BlockSpec((2, BLK, HID), lambda i: (0, i, 0)),
        out_shape=jax.ShapeDtypeStruct((2, NPAD, HID), jnp.float32),
    )(S, C4, Z, Wnu, Wsu, bu, Wni, Wsi, bi)


def _dense_final(S, C4, Z, Wnu, Wsu, bu, Wni, Wsi, bi, Wp2, bp2):
    wspec = pl.BlockSpec((HID, HID), lambda i: (0, 0))
    bspec = pl.BlockSpec((1, HID), lambda i: (0, 0))
    return pl.pallas_call(
        _dense_final_body,
        grid=(NBLK,),
        in_specs=[
            pl.BlockSpec((2, BLK, HID), lambda i: (0, i, 0)),
            pl.BlockSpec((2, 1, 1, BLK), lambda i: (0, i, 0, 0)),
            pl.BlockSpec((2, BLK, HID), lambda i: (0, i, 0)),
            wspec, wspec, bspec, wspec, wspec, bspec,
            pl.BlockSpec((2, HID), lambda i: (0, 0)),
            bspec,
        ],
        out_specs=pl.BlockSpec((1, 1, BLK), lambda i: (i, 0, 0)),
        out_shape=jax.ShapeDtypeStruct((NBLK, 1, BLK), jnp.float32),
    )(S, C4, Z, Wnu, Wsu, bu, Wni, Wsi, bi, Wp2, bp2)


def kernel(user_ids, item_ids, edge_index_u2i, edge_index_i2u,
           user_table, item_table,
           Wn1_u2i, Ws1_u2i, b1_u2i, Wn1_i2u, Ws1_i2u, b1_i2u,
           Wn2_u2i, Ws2_u2i, b2_u2i, Wn2_i2u, Ws2_i2u, b2_i2u,
           Wp, bp):
    idpad = jnp.zeros((NPAD - NU,), jnp.int32)
    ids_all = jnp.concatenate([user_ids, idpad, item_ids, idpad])
    epad = EPAD - E
    src_all = jnp.concatenate([
        edge_index_u2i[0], jnp.zeros((epad,), jnp.int32),
        edge_index_i2u[0] + NPAD, jnp.full((epad,), NPAD, jnp.int32),
    ])
    dstfill = jnp.full((epad,), NPAD - 1, jnp.int32)
    dst_all = jnp.concatenate([
        edge_index_u2i[1], dstfill,
        edge_index_i2u[1], dstfill,
    ])

    Zf, counts = _embed_hist(ids_all, user_table, item_table, dst_all)
    Z = Zf.reshape(2, NPAD, EMB)
    C4 = counts.reshape(2, NBLK, 1, BLK)

    b1u = b1_i2u.reshape(1, HID)
    b1i = b1_u2i.reshape(1, HID)
    b2u = b2_i2u.reshape(1, HID)
    b2i = b2_u2i.reshape(1, HID)
    Wp2 = Wp.reshape(2, HID)
    bp2 = jnp.broadcast_to(bp.reshape(1, 1), (1, HID))

    S1 = _segsum(Zf, src_all, dst_all).reshape(2, NPAD, EMB)
    H1 = _dense_layer(S1, C4, Z, Wn1_i2u, Ws1_i2u, b1u, Wn1_u2i, Ws1_u2i, b1i)
    S2 = _segsum(H1.reshape(2 * NPAD, HID), src_all, dst_all).reshape(2, NPAD, HID)
    out = _dense_final(S2, C4, H1, Wn2_i2u, Ws2_i2u, b2u,
                       Wn2_u2i, Ws2_u2i, b2i, Wp2, bp2)
    return out.reshape(NPAD, 1)[:NU]


# speedup vs baseline: 1.5366x; 1.5366x over previous
import jax
import jax.numpy as jnp
from jax import lax
from jax.experimental import pallas as pl
from jax.experimental.pallas import tpu as pltpu
from jax.experimental.pallas import tpu_sc as plsc

NU = 10000
NI = 10000
E = 320000
EMB = 128
HID = 128

NPAD = 10240
RPT = NPAD // 16
K = 128
CHUNKS = 157
EPT = CHUNKS * K
EPAD = EPT * 16
NBUF = 2
NBLK = 40
BLK = NPAD // NBLK
GPW = NPAD // 32
GK = 80


def _embed_hist_body(ids_ref, utab_ref, itab_ref, dst_ref,
                     z_ref, cnt_ref,
                     idx_v, gidx_v, grow_v, ones_v, zrow_v, cacc_sh, sem):
    c = lax.axis_index("c")
    s = lax.axis_index("s")
    wid = c * 16 + s
    zeros16 = jnp.zeros((16,), jnp.float32)
    ones16 = jnp.ones((16,), jnp.float32)

    def obody(i, _):
        ones_v[pl.ds(i * 16, 16)] = ones16
        return 0
    lax.fori_loop(0, K // 16, obody, 0)

    def zcbody(i, _):
        zrow_v[pl.ds(i * 16, 16)] = zeros16
        return 0
    lax.fori_loop(0, RPT // 16, zcbody, 0)
    pltpu.sync_copy(zrow_v, cacc_sh.at[pl.ds(s * RPT, RPT)])
    plsc.subcore_barrier()

    for j in range(GPW // GK):
        base = wid * GPW + j * GK
        pltpu.sync_copy(ids_ref.at[pl.ds(base, GK)], gidx_v)
        pltpu.async_copy(utab_ref.at[gidx_v], grow_v, sem).wait()
        pltpu.sync_copy(grow_v, z_ref.at[pl.ds(base, GK)])
    for j in range(GPW // GK):
        base = NPAD + wid * GPW + j * GK
        pltpu.sync_copy(ids_ref.at[pl.ds(base, GK)], gidx_v)
        pltpu.async_copy(itab_ref.at[gidx_v], grow_v, sem).wait()
        pltpu.sync_copy(grow_v, z_ref.at[pl.ds(base, GK)])

    def hbody(j, _):
        off = c * EPAD + s * EPT + j * K
        pltpu.sync_copy(dst_ref.at[pl.ds(off, K)], idx_v)
        pltpu.sync_copy(ones_v, cacc_sh.at[idx_v], add=True)
        return 0
    lax.fori_loop(0, CHUNKS, hbody, 0)

    plsc.subcore_barrier()
    pltpu.sync_copy(cacc_sh.at[pl.ds(s * RPT, RPT)],
                    cnt_ref.at[pl.ds(c * NPAD + s * RPT, RPT)])


def _segsum_body(t_ref, src_ref, dst_ref, s_out_ref,
                 isrc_v, idst_v, rows_v, acc_sh, sem):
    c = lax.axis_index("c")
    s = lax.axis_index("s")
    ebase = c * EPAD + s * EPT
    zeros16 = jnp.zeros((16,), jnp.float32)

    def zbody(i, _):
        for k in range(EMB // 16):
            rows_v[i, pl.ds(k * 16, 16)] = zeros16
        return 0
    lax.fori_loop(0, K, zbody, 0)
    for j in range(RPT // K):
        pltpu.sync_copy(rows_v, acc_sh.at[pl.ds(s * RPT + j * K, K)])
    plsc.subcore_barrier()

    def body(j, _):
        off = ebase + j * K
        pltpu.sync_copy(src_ref.at[pl.ds(off, K)], isrc_v)
        pltpu.sync_copy(dst_ref.at[pl.ds(off, K)], idst_v)
        pltpu.async_copy(t_ref.at[isrc_v], rows_v, sem).wait()
        pltpu.sync_copy(rows_v, acc_sh.at[idst_v], add=True)
        return 0
    lax.fori_loop(0, CHUNKS, body, 0)

    plsc.subcore_barrier()
    pltpu.sync_copy(acc_sh.at[pl.ds(s * RPT, RPT)],
                    s_out_ref.at[pl.ds(c * NPAD + s * RPT, RPT)])


def _dense_layer_body(s_ref, c_ref, z_ref,
                      wnu_ref, wsu_ref, bu_ref, wni_ref, wsi_ref, bi_ref,
                      h_ref):
    cu = jnp.clip(c_ref[1, 0, 0, :], 1.0, None)
    ci = jnp.clip(c_ref[0, 0, 0, :], 1.0, None)
    agg_u = s_ref[1] / cu[:, None]
    agg_i = s_ref[0] / ci[:, None]
    nu = (jnp.dot(agg_u, wnu_ref[...], preferred_element_type=jnp.float32)
          + jnp.dot(z_ref[0], wsu_ref[...], preferred_element_type=jnp.float32)
          + bu_ref[0, :])
    ni = (jnp.dot(agg_i, wni_ref[...], preferred_element_type=jnp.float32)
          + jnp.dot(z_ref[1], wsi_ref[...], preferred_element_type=jnp.float32)
          + bi_ref[0, :])
    h_ref[0] = jnp.maximum(nu, 0.0)
    h_ref[1] = jnp.maximum(ni, 0.0)


def _dense_final_body(s_ref, c_ref, z_ref,
                      wnu_ref, wsu_ref, bu_ref, wni_ref, wsi_ref, bi_ref,
                      wp_ref, bp_ref,
                      out_ref):
    cu = jnp.clip(c_ref[1, 0, 0, :], 1.0, None)
    ci = jnp.clip(c_ref[0, 0, 0, :], 1.0, None)
    agg_u = s_ref[1] / cu[:, None]
    agg_i = s_ref[0] / ci[:, None]
    nu = (jnp.dot(agg_u, wnu_ref[...], preferred_element_type=jnp.float32)
          + jnp.dot(z_ref[0], wsu_ref[...], preferred_element_type=jnp.float32)
          + bu_ref[0, :])
    ni = (jnp.dot(agg_i, wni_ref[...], preferred_element_type=jnp.float32)
          + jnp.dot(z_ref[1], wsi_ref[...], preferred_element_type=jnp.float32)
          + bi_ref[0, :])
    hu = jnp.maximum(nu, 0.0)
    hi = jnp.maximum(ni, 0.0)
    res = (jnp.sum(hu * wp_ref[0, :][None, :], axis=1)
           + jnp.sum(hi * wp_ref[1, :][None, :], axis=1)
           + bp_ref[0, 0])
    out_ref[0, 0, :] = res


def _sc_mesh():
    return plsc.VectorSubcoreMesh(core_axis_name="c", subcore_axis_name="s")


def _embed_hist(ids_all, user_table, item_table, dst_all):
    return pl.kernel(
        _embed_hist_body,
        out_type=(
            jax.ShapeDtypeStruct((2 * NPAD, EMB), jnp.float32),
            jax.ShapeDtypeStruct((2 * NPAD,), jnp.float32),
        ),
        mesh=_sc_mesh(),
        scratch_types=[
            pltpu.VMEM((K,), jnp.int32),
            pltpu.VMEM((GK,), jnp.int32),
            pltpu.VMEM((GK, EMB), jnp.float32),
            pltpu.VMEM((K,), jnp.float32),
            pltpu.VMEM((RPT,), jnp.float32),
            pltpu.VMEM_SHARED((NPAD,), jnp.float32),
            pltpu.SemaphoreType.DMA,
        ],
    )(ids_all, user_table, item_table, dst_all)


def _segsum(table_flat, src_all, dst_all):
    return pl.kernel(
        _segsum_body,
        out_type=jax.ShapeDtypeStruct((2 * NPAD, EMB), jnp.float32),
        mesh=_sc_mesh(),
        scratch_types=[
            pltpu.VMEM((K,), jnp.int32),
            pltpu.VMEM((K,), jnp.int32),
            pltpu.VMEM((K, EMB), jnp.float32),
            pltpu.VMEM_SHARED((NPAD, EMB), jnp.float32),
            pltpu.SemaphoreType.DMA,
        ],
    )(table_flat, src_all, dst_all)


def _dense_layer(S, C4, Z, Wnu, Wsu, bu, Wni, Wsi, bi):
    wspec = pl.BlockSpec((EMB, HID), lambda i: (0, 0))
    bspec = pl.BlockSpec((1, HID), lambda i: (0, 0))
    return pl.pallas_call(
        _dense_layer_body,
        grid=(NBLK,),
        in_specs=[
            pl.BlockSpec((2, BLK, EMB), lambda i: (0, i, 0)),
            pl.BlockSpec((2, 1, 1, BLK), lambda i: (0, i, 0, 0)),
            pl.BlockSpec((2, BLK, EMB), lambda i: (0, i, 0)),
            wspec, wspec, bspec, wspec, wspec, bspec,
        ],
        out_specs=pl.BlockSpec((2, BLK, HID), lambda i: (0, i, 0)),
        out_shape=jax.ShapeDtypeStruct((2, NPAD, HID), jnp.float32),
    )(S, C4, Z, Wnu, Wsu, bu, Wni, Wsi, bi)


def _dense_final(S, C4, Z, Wnu, Wsu, bu, Wni, Wsi, bi, Wp2, bp2):
    wspec = pl.BlockSpec((HID, HID), lambda i: (0, 0))
    bspec = pl.BlockSpec((1, HID), lambda i: (0, 0))
    return pl.pallas_call(
        _dense_final_body,
        grid=(NBLK,),
        in_specs=[
            pl.BlockSpec((2, BLK, HID), lambda i: (0, i, 0)),
            pl.BlockSpec((2, 1, 1, BLK), lambda i: (0, i, 0, 0)),
            pl.BlockSpec((2, BLK, HID), lambda i: (0, i, 0)),
            wspec, wspec, bspec, wspec, wspec, bspec,
            pl.BlockSpec((2, HID), lambda i: (0, 0)),
            bspec,
        ],
        out_specs=pl.BlockSpec((1, 1, BLK), lambda i: (i, 0, 0)),
        out_shape=jax.ShapeDtypeStruct((NBLK, 1, BLK), jnp.float32),
    )(S, C4, Z, Wnu, Wsu, bu, Wni, Wsi, bi, Wp2, bp2)


def kernel(user_ids, item_ids, edge_index_u2i, edge_index_i2u,
           user_table, item_table,
           Wn1_u2i, Ws1_u2i, b1_u2i, Wn1_i2u, Ws1_i2u, b1_i2u,
           Wn2_u2i, Ws2_u2i, b2_u2i, Wn2_i2u, Ws2_i2u, b2_i2u,
           Wp, bp):
    idpad = jnp.zeros((NPAD - NU,), jnp.int32)
    ids_all = jnp.concatenate([user_ids, idpad, item_ids, idpad])
    epad = EPAD - E
    src_all = jnp.concatenate([
        edge_index_u2i[0], jnp.zeros((epad,), jnp.int32),
        edge_index_i2u[0] + NPAD, jnp.full((epad,), NPAD, jnp.int32),
    ])
    dstfill = jnp.full((epad,), NPAD - 1, jnp.int32)
    dst_all = jnp.concatenate([
        edge_index_u2i[1], dstfill,
        edge_index_i2u[1], dstfill,
    ])

    Zf, counts = _embed_hist(ids_all, user_table, item_table, dst_all)
    Z = Zf.reshape(2, NPAD, EMB)
    C4 = counts.reshape(2, NBLK, 1, BLK)

    b1u = b1_i2u.reshape(1, HID)
    b1i = b1_u2i.reshape(1, HID)
    b2u = b2_i2u.reshape(1, HID)
    b2i = b2_u2i.reshape(1, HID)
    Wp2 = Wp.reshape(2, HID)
    bp2 = jnp.broadcast_to(bp.reshape(1, 1), (1, HID))

    S1 = _segsum(Zf, src_all, dst_all).reshape(2, NPAD, EMB)
    H1 = _dense_layer(S1, C4, Z, Wn1_i2u, Ws1_i2u, b1u, Wn1_u2i, Ws1_u2i, b1i)
    S2 = _segsum(H1.reshape(2 * NPAD, HID), src_all, dst_all).reshape(2, NPAD, HID)
    out = _dense_final(S2, C4, H1, Wn2_i2u, Ws2_i2u, b2u,
                       Wn2_u2i, Ws2_u2i, b2i, Wp2, bp2)
    return out.reshape(NPAD, 1)[:NU]


# speedup vs baseline: 1.6590x; 1.0797x over previous
import jax
import jax.numpy as jnp
from jax import lax
from jax.experimental import pallas as pl
from jax.experimental.pallas import tpu as pltpu
from jax.experimental.pallas import tpu_sc as plsc

NU = 10000
NI = 10000
E = 320000
EMB = 128
HID = 128

NPAD = 10240
RPT = NPAD // 16
K = 128
CHUNKS = 157
EPT = CHUNKS * K
EPAD = EPT * 16
NBUF = 2
NBLK = 40
BLK = NPAD // NBLK
GPW = NPAD // 32
GK = 80


def _embed_hist_body(ids_ref, utab_ref, itab_ref, dst_ref,
                     z_ref, cnt_ref,
                     idx_v, gidx_v, grow_v, ones_v, zrow_v, cacc_sh, sem):
    c = lax.axis_index("c")
    s = lax.axis_index("s")
    wid = c * 16 + s
    zeros16 = jnp.zeros((16,), jnp.float32)
    ones16 = jnp.ones((16,), jnp.float32)

    def obody(i, _):
        ones_v[pl.ds(i * 16, 16)] = ones16
        return 0
    lax.fori_loop(0, K // 16, obody, 0)

    def zcbody(i, _):
        zrow_v[pl.ds(i * 16, 16)] = zeros16
        return 0
    lax.fori_loop(0, RPT // 16, zcbody, 0)
    pltpu.sync_copy(zrow_v, cacc_sh.at[pl.ds(s * RPT, RPT)])
    plsc.subcore_barrier()

    for j in range(GPW // GK):
        base = wid * GPW + j * GK
        pltpu.sync_copy(ids_ref.at[pl.ds(base, GK)], gidx_v)
        pltpu.async_copy(utab_ref.at[gidx_v], grow_v, sem).wait()
        pltpu.sync_copy(grow_v, z_ref.at[pl.ds(base, GK)])
    for j in range(GPW // GK):
        base = NPAD + wid * GPW + j * GK
        pltpu.sync_copy(ids_ref.at[pl.ds(base, GK)], gidx_v)
        pltpu.async_copy(itab_ref.at[gidx_v], grow_v, sem).wait()
        pltpu.sync_copy(grow_v, z_ref.at[pl.ds(base, GK)])

    def hbody(j, _):
        off = c * EPAD + s * EPT + j * K
        pltpu.sync_copy(dst_ref.at[pl.ds(off, K)], idx_v)
        pltpu.sync_copy(ones_v, cacc_sh.at[idx_v], add=True)
        return 0
    lax.fori_loop(0, CHUNKS, hbody, 0)

    plsc.subcore_barrier()
    pltpu.sync_copy(cacc_sh.at[pl.ds(s * RPT, RPT)],
                    cnt_ref.at[pl.ds(c * NPAD + s * RPT, RPT)])


def _segsum_body(t_ref, src_ref, dst_ref, s_out_ref,
                 isrc_v, idst_v, rows_v, acc_sh, sem):
    c = lax.axis_index("c")
    s = lax.axis_index("s")
    ebase = c * EPAD + s * EPT
    zeros16 = jnp.zeros((16,), jnp.float32)

    def zbody(i, _):
        for k in range(EMB // 16):
            rows_v[i, pl.ds(k * 16, 16)] = zeros16
        return 0
    lax.fori_loop(0, K, zbody, 0)
    for j in range(RPT // K):
        pltpu.sync_copy(rows_v, acc_sh.at[pl.ds(s * RPT + j * K, K)])
    plsc.subcore_barrier()

    def body(j, _):
        off = ebase + j * K
        pltpu.sync_copy(src_ref.at[pl.ds(off, K)], isrc_v)
        pltpu.sync_copy(dst_ref.at[pl.ds(off, K)], idst_v)
        pltpu.async_copy(t_ref.at[isrc_v], rows_v, sem).wait()
        pltpu.sync_copy(rows_v, acc_sh.at[idst_v], add=True)
        return 0
    lax.fori_loop(0, CHUNKS, body, 0)

    plsc.subcore_barrier()
    pltpu.sync_copy(acc_sh.at[pl.ds(s * RPT, RPT)],
                    s_out_ref.at[pl.ds(c * NPAD + s * RPT, RPT)])


def _dense_layer_body(s_ref, c_ref, z_ref,
                      wnu_ref, wsu_ref, bu_ref, wni_ref, wsi_ref, bi_ref,
                      h_ref):
    cu = jnp.clip(c_ref[1, 0, 0, :], 1.0, None)
    ci = jnp.clip(c_ref[0, 0, 0, :], 1.0, None)
    agg_u = s_ref[1] / cu[:, None]
    agg_i = s_ref[0] / ci[:, None]
    nu = (jnp.dot(agg_u, wnu_ref[...], preferred_element_type=jnp.float32)
          + jnp.dot(z_ref[0], wsu_ref[...], preferred_element_type=jnp.float32)
          + bu_ref[0, :])
    ni = (jnp.dot(agg_i, wni_ref[...], preferred_element_type=jnp.float32)
          + jnp.dot(z_ref[1], wsi_ref[...], preferred_element_type=jnp.float32)
          + bi_ref[0, :])
    h_ref[0] = jnp.maximum(nu, 0.0)
    h_ref[1] = jnp.maximum(ni, 0.0)


def _dense_final_body(s_ref, c_ref, z_ref,
                      wnu_ref, wsu_ref, bu_ref, wni_ref, wsi_ref, bi_ref,
                      wp_ref, bp_ref,
                      out_ref):
    cu = jnp.clip(c_ref[1, 0, 0, :], 1.0, None)
    ci = jnp.clip(c_ref[0, 0, 0, :], 1.0, None)
    agg_u = s_ref[1] / cu[:, None]
    agg_i = s_ref[0] / ci[:, None]
    nu = (jnp.dot(agg_u, wnu_ref[...], preferred_element_type=jnp.float32)
          + jnp.dot(z_ref[0], wsu_ref[...], preferred_element_type=jnp.float32)
          + bu_ref[0, :])
    ni = (jnp.dot(agg_i, wni_ref[...], preferred_element_type=jnp.float32)
          + jnp.dot(z_ref[1], wsi_ref[...], preferred_element_type=jnp.float32)
          + bi_ref[0, :])
    hu = jnp.maximum(nu, 0.0)
    hi = jnp.maximum(ni, 0.0)
    res = (jnp.sum(hu * wp_ref[0, :][None, :], axis=1)
           + jnp.sum(hi * wp_ref[1, :][None, :], axis=1)
           + bp_ref[0, 0])
    out_ref[0, 0, :] = res


def _sc_mesh():
    return plsc.VectorSubcoreMesh(core_axis_name="c", subcore_axis_name="s")


def _embed_hist(ids_all, user_table, item_table, dst_all):
    return pl.kernel(
        _embed_hist_body,
        out_type=(
            jax.ShapeDtypeStruct((2 * NPAD, EMB), jnp.float32),
            jax.ShapeDtypeStruct((2 * NPAD,), jnp.float32),
        ),
        mesh=_sc_mesh(),
        scratch_types=[
            pltpu.VMEM((K,), jnp.int32),
            pltpu.VMEM((GK,), jnp.int32),
            pltpu.VMEM((GK, EMB), jnp.float32),
            pltpu.VMEM((K,), jnp.float32),
            pltpu.VMEM((RPT,), jnp.float32),
            pltpu.VMEM_SHARED((NPAD,), jnp.float32),
            pltpu.SemaphoreType.DMA,
        ],
    )(ids_all, user_table, item_table, dst_all)


def _segsum(table_flat, src_all, dst_all):
    return pl.kernel(
        _segsum_body,
        out_type=jax.ShapeDtypeStruct((2 * NPAD, EMB), jnp.float32),
        mesh=_sc_mesh(),
        scratch_types=[
            pltpu.VMEM((K,), jnp.int32),
            pltpu.VMEM((K,), jnp.int32),
            pltpu.VMEM((K, EMB), jnp.float32),
            pltpu.VMEM_SHARED((NPAD, EMB), jnp.float32),
            pltpu.SemaphoreType.DMA,
        ],
    )(table_flat, src_all, dst_all)


def _dense_layer(S, C4, Z, Wnu, Wsu, bu, Wni, Wsi, bi):
    wspec = pl.BlockSpec((EMB, HID), lambda i: (0, 0))
    bspec = pl.BlockSpec((1, HID), lambda i: (0, 0))
    return pl.pallas_call(
        _dense_layer_body,
        grid=(NBLK,),
        in_specs=[
            pl.BlockSpec((2, BLK, EMB), lambda i: (0, i, 0)),
            pl.BlockSpec((2, 1, 1, BLK), lambda i: (0, i, 0, 0)),
            pl.BlockSpec((2, BLK, EMB), lambda i: (0, i, 0)),
            wspec, wspec, bspec, wspec, wspec, bspec,
        ],
        out_specs=pl.BlockSpec((2, BLK, HID), lambda i: (0, i, 0)),
        out_shape=jax.ShapeDtypeStruct((2, NPAD, HID), jnp.float32),
    )(S, C4, Z, Wnu, Wsu, bu, Wni, Wsi, bi)


def _dense_final(S, C4, Z, Wnu, Wsu, bu, Wni, Wsi, bi, Wp2, bp2):
    wspec = pl.BlockSpec((HID, HID), lambda i: (0, 0))
    bspec = pl.BlockSpec((1, HID), lambda i: (0, 0))
    return pl.pallas_call(
        _dense_final_body,
        grid=(NBLK,),
        in_specs=[
            pl.BlockSpec((2, BLK, HID), lambda i: (0, i, 0)),
            pl.BlockSpec((2, 1, 1, BLK), lambda i: (0, i, 0, 0)),
            pl.BlockSpec((2, BLK, HID), lambda i: (0, i, 0)),
            wspec, wspec, bspec, wspec, wspec, bspec,
            pl.BlockSpec((2, HID), lambda i: (0, 0)),
            bspec,
        ],
        out_specs=pl.BlockSpec((1, 1, BLK), lambda i: (i, 0, 0)),
        out_shape=jax.ShapeDtypeStruct((NBLK, 1, BLK), jnp.float32),
    )(S, C4, Z, Wnu, Wsu, bu, Wni, Wsi, bi, Wp2, bp2)


def kernel(user_ids, item_ids, edge_index_u2i, edge_index_i2u,
           user_table, item_table,
           Wn1_u2i, Ws1_u2i, b1_u2i, Wn1_i2u, Ws1_i2u, b1_i2u,
           Wn2_u2i, Ws2_u2i, b2_u2i, Wn2_i2u, Ws2_i2u, b2_i2u,
           Wp, bp):
    idpad = jnp.zeros((NPAD - NU,), jnp.int32)
    ids_all = jnp.concatenate([user_ids, idpad, item_ids, idpad])
    epad = EPAD - E
    srcfill = jnp.arange(epad, dtype=jnp.int32) % NPAD
    dstfill = NU + (jnp.arange(epad, dtype=jnp.int32) % (NPAD - NU))
    src_all = jnp.concatenate([
        edge_index_u2i[0], srcfill,
        edge_index_i2u[0] + NPAD, srcfill + NPAD,
    ])
    dst_all = jnp.concatenate([
        edge_index_u2i[1], dstfill,
        edge_index_i2u[1], dstfill,
    ])

    Zf, counts = _embed_hist(ids_all, user_table, item_table, dst_all)
    Z = Zf.reshape(2, NPAD, EMB)
    C4 = counts.reshape(2, NBLK, 1, BLK)

    b1u = b1_i2u.reshape(1, HID)
    b1i = b1_u2i.reshape(1, HID)
    b2u = b2_i2u.reshape(1, HID)
    b2i = b2_u2i.reshape(1, HID)
    Wp2 = Wp.reshape(2, HID)
    bp2 = jnp.broadcast_to(bp.reshape(1, 1), (1, HID))

    S1 = _segsum(Zf, src_all, dst_all).reshape(2, NPAD, EMB)
    H1 = _dense_layer(S1, C4, Z, Wn1_i2u, Ws1_i2u, b1u, Wn1_u2i, Ws1_u2i, b1i)
    S2 = _segsum(H1.reshape(2 * NPAD, HID), src_all, dst_all).reshape(2, NPAD, HID)
    out = _dense_final(S2, C4, H1, Wn2_i2u, Ws2_i2u, b2u,
                       Wn2_u2i, Ws2_u2i, b2i, Wp2, bp2)
    return out.reshape(NPAD, 1)[:NU]


# speedup vs baseline: 2.1807x; 1.3145x over previous
import jax
import jax.numpy as jnp
from jax import lax
from jax.experimental import pallas as pl
from jax.experimental.pallas import tpu as pltpu
from jax.experimental.pallas import tpu_sc as plsc

NU = 10000
NI = 10000
E = 320000
EMB = 128
HID = 128

NPAD = 10240
RPT = NPAD // 16
K = 128
CHUNKS = 158
EPT = CHUNKS * K
EPAD = EPT * 16
NBUF = 2
NBLK = 40
BLK = NPAD // NBLK
GPW = NPAD // 32
GK = 80


def _embed_hist_body(ids_ref, utab_ref, itab_ref, dst_ref,
                     z_ref, cnt_ref,
                     idx_v, gidx_v, grow_v, ones_v, zrow_v, cacc_sh, sem):
    c = lax.axis_index("c")
    s = lax.axis_index("s")
    wid = c * 16 + s
    zeros16 = jnp.zeros((16,), jnp.float32)
    ones16 = jnp.ones((16,), jnp.float32)

    def obody(i, _):
        ones_v[pl.ds(i * 16, 16)] = ones16
        return 0
    lax.fori_loop(0, K // 16, obody, 0)

    def zcbody(i, _):
        zrow_v[pl.ds(i * 16, 16)] = zeros16
        return 0
    lax.fori_loop(0, RPT // 16, zcbody, 0)
    pltpu.sync_copy(zrow_v, cacc_sh.at[pl.ds(s * RPT, RPT)])
    plsc.subcore_barrier()

    for j in range(GPW // GK):
        base = wid * GPW + j * GK
        pltpu.sync_copy(ids_ref.at[pl.ds(base, GK)], gidx_v)
        pltpu.async_copy(utab_ref.at[gidx_v], grow_v, sem).wait()
        pltpu.sync_copy(grow_v, z_ref.at[pl.ds(base, GK)])
    for j in range(GPW // GK):
        base = NPAD + wid * GPW + j * GK
        pltpu.sync_copy(ids_ref.at[pl.ds(base, GK)], gidx_v)
        pltpu.async_copy(itab_ref.at[gidx_v], grow_v, sem).wait()
        pltpu.sync_copy(grow_v, z_ref.at[pl.ds(base, GK)])

    def hbody(j, _):
        off = c * EPAD + s * EPT + j * K
        pltpu.sync_copy(dst_ref.at[pl.ds(off, K)], idx_v)
        pltpu.sync_copy(ones_v, cacc_sh.at[idx_v], add=True)
        return 0
    lax.fori_loop(0, CHUNKS, hbody, 0)

    plsc.subcore_barrier()
    pltpu.sync_copy(cacc_sh.at[pl.ds(s * RPT, RPT)],
                    cnt_ref.at[pl.ds(c * NPAD + s * RPT, RPT)])


def _segsum_body(t_ref, src_ref, dst_ref, s_out_ref,
                 isrc_v, idst_v, rows_v, isrc2_v, idst2_v, rows2_v,
                 acc_sh, sem, sem2, ssem, ssem2):
    c = lax.axis_index("c")
    s = lax.axis_index("s")
    ebase = c * EPAD + s * EPT
    zeros16 = jnp.zeros((16,), jnp.float32)

    def zbody(i, _):
        for k in range(EMB // 16):
            rows_v[i, pl.ds(k * 16, 16)] = zeros16
        return 0
    lax.fori_loop(0, K, zbody, 0)
    for j in range(RPT // K):
        pltpu.sync_copy(rows_v, acc_sh.at[pl.ds(s * RPT + j * K, K)])
    plsc.subcore_barrier()

    def body(pr, _):
        off0 = ebase + (2 * pr) * K
        off1 = off0 + K
        pltpu.sync_copy(src_ref.at[pl.ds(off0, K)], isrc_v)
        pltpu.sync_copy(dst_ref.at[pl.ds(off0, K)], idst_v)
        g0 = pltpu.async_copy(t_ref.at[isrc_v], rows_v, sem)
        pltpu.sync_copy(src_ref.at[pl.ds(off1, K)], isrc2_v)
        pltpu.sync_copy(dst_ref.at[pl.ds(off1, K)], idst2_v)
        g1 = pltpu.async_copy(t_ref.at[isrc2_v], rows2_v, sem2)
        g0.wait()
        s0 = pltpu.async_copy(rows_v, acc_sh.at[idst_v], ssem, add=True)
        g1.wait()
        s1 = pltpu.async_copy(rows2_v, acc_sh.at[idst2_v], ssem2, add=True)
        s0.wait()
        s1.wait()
        return 0
    lax.fori_loop(0, CHUNKS // 2, body, 0)

    plsc.subcore_barrier()
    pltpu.sync_copy(acc_sh.at[pl.ds(s * RPT, RPT)],
                    s_out_ref.at[pl.ds(c * NPAD + s * RPT, RPT)])


def _dense_layer_body(s_ref, c_ref, z_ref,
                      wnu_ref, wsu_ref, bu_ref, wni_ref, wsi_ref, bi_ref,
                      h_ref):
    cu = jnp.clip(c_ref[1, 0, 0, :], 1.0, None)
    ci = jnp.clip(c_ref[0, 0, 0, :], 1.0, None)
    agg_u = s_ref[1] / cu[:, None]
    agg_i = s_ref[0] / ci[:, None]
    nu = (jnp.dot(agg_u, wnu_ref[...], preferred_element_type=jnp.float32)
          + jnp.dot(z_ref[0], wsu_ref[...], preferred_element_type=jnp.float32)
          + bu_ref[0, :])
    ni = (jnp.dot(agg_i, wni_ref[...], preferred_element_type=jnp.float32)
          + jnp.dot(z_ref[1], wsi_ref[...], preferred_element_type=jnp.float32)
          + bi_ref[0, :])
    h_ref[0] = jnp.maximum(nu, 0.0)
    h_ref[1] = jnp.maximum(ni, 0.0)


def _dense_final_body(s_ref, c_ref, z_ref,
                      wnu_ref, wsu_ref, bu_ref, wni_ref, wsi_ref, bi_ref,
                      wp_ref, bp_ref,
                      out_ref):
    cu = jnp.clip(c_ref[1, 0, 0, :], 1.0, None)
    ci = jnp.clip(c_ref[0, 0, 0, :], 1.0, None)
    agg_u = s_ref[1] / cu[:, None]
    agg_i = s_ref[0] / ci[:, None]
    nu = (jnp.dot(agg_u, wnu_ref[...], preferred_element_type=jnp.float32)
          + jnp.dot(z_ref[0], wsu_ref[...], preferred_element_type=jnp.float32)
          + bu_ref[0, :])
    ni = (jnp.dot(agg_i, wni_ref[...], preferred_element_type=jnp.float32)
          + jnp.dot(z_ref[1], wsi_ref[...], preferred_element_type=jnp.float32)
          + bi_ref[0, :])
    hu = jnp.maximum(nu, 0.0)
    hi = jnp.maximum(ni, 0.0)
    res = (jnp.sum(hu * wp_ref[0, :][None, :], axis=1)
           + jnp.sum(hi * wp_ref[1, :][None, :], axis=1)
           + bp_ref[0, 0])
    out_ref[0, 0, :] = res


def _sc_mesh():
    return plsc.VectorSubcoreMesh(core_axis_name="c", subcore_axis_name="s")


def _embed_hist(ids_all, user_table, item_table, dst_all):
    return pl.kernel(
        _embed_hist_body,
        out_type=(
            jax.ShapeDtypeStruct((2 * NPAD, EMB), jnp.float32),
            jax.ShapeDtypeStruct((2 * NPAD,), jnp.float32),
        ),
        mesh=_sc_mesh(),
        scratch_types=[
            pltpu.VMEM((K,), jnp.int32),
            pltpu.VMEM((GK,), jnp.int32),
            pltpu.VMEM((GK, EMB), jnp.float32),
            pltpu.VMEM((K,), jnp.float32),
            pltpu.VMEM((RPT,), jnp.float32),
            pltpu.VMEM_SHARED((NPAD,), jnp.float32),
            pltpu.SemaphoreType.DMA,
        ],
    )(ids_all, user_table, item_table, dst_all)


def _segsum(table_flat, src_all, dst_all):
    return pl.kernel(
        _segsum_body,
        out_type=jax.ShapeDtypeStruct((2 * NPAD, EMB), jnp.float32),
        mesh=_sc_mesh(),
        scratch_types=[
            pltpu.VMEM((K,), jnp.int32),
            pltpu.VMEM((K,), jnp.int32),
            pltpu.VMEM((K, EMB), jnp.float32),
            pltpu.VMEM((K,), jnp.int32),
            pltpu.VMEM((K,), jnp.int32),
            pltpu.VMEM((K, EMB), jnp.float32),
            pltpu.VMEM_SHARED((NPAD, EMB), jnp.float32),
            pltpu.SemaphoreType.DMA,
            pltpu.SemaphoreType.DMA,
            pltpu.SemaphoreType.DMA,
            pltpu.SemaphoreType.DMA,
        ],
    )(table_flat, src_all, dst_all)


def _dense_layer(S, C4, Z, Wnu, Wsu, bu, Wni, Wsi, bi):
    wspec = pl.BlockSpec((EMB, HID), lambda i: (0, 0))
    bspec = pl.BlockSpec((1, HID), lambda i: (0, 0))
    return pl.pallas_call(
        _dense_layer_body,
        grid=(NBLK,),
        in_specs=[
            pl.BlockSpec((2, BLK, EMB), lambda i: (0, i, 0)),
            pl.BlockSpec((2, 1, 1, BLK), lambda i: (0, i, 0, 0)),
            pl.BlockSpec((2, BLK, EMB), lambda i: (0, i, 0)),
            wspec, wspec, bspec, wspec, wspec, bspec,
        ],
        out_specs=pl.BlockSpec((2, BLK, HID), lambda i: (0, i, 0)),
        out_shape=jax.ShapeDtypeStruct((2, NPAD, HID), jnp.float32),
    )(S, C4, Z, Wnu, Wsu, bu, Wni, Wsi, bi)


def _dense_final(S, C4, Z, Wnu, Wsu, bu, Wni, Wsi, bi, Wp2, bp2):
    wspec = pl.BlockSpec((HID, HID), lambda i: (0, 0))
    bspec = pl.BlockSpec((1, HID), lambda i: (0, 0))
    return pl.pallas_call(
        _dense_final_body,
        grid=(NBLK,),
        in_specs=[
            pl.BlockSpec((2, BLK, HID), lambda i: (0, i, 0)),
            pl.BlockSpec((2, 1, 1, BLK), lambda i: (0, i, 0, 0)),
            pl.BlockSpec((2, BLK, HID), lambda i: (0, i, 0)),
            wspec, wspec, bspec, wspec, wspec, bspec,
            pl.BlockSpec((2, HID), lambda i: (0, 0)),
            bspec,
        ],
        out_specs=pl.BlockSpec((1, 1, BLK), lambda i: (i, 0, 0)),
        out_shape=jax.ShapeDtypeStruct((NBLK, 1, BLK), jnp.float32),
    )(S, C4, Z, Wnu, Wsu, bu, Wni, Wsi, bi, Wp2, bp2)


def kernel(user_ids, item_ids, edge_index_u2i, edge_index_i2u,
           user_table, item_table,
           Wn1_u2i, Ws1_u2i, b1_u2i, Wn1_i2u, Ws1_i2u, b1_i2u,
           Wn2_u2i, Ws2_u2i, b2_u2i, Wn2_i2u, Ws2_i2u, b2_i2u,
           Wp, bp):
    idpad = jnp.zeros((NPAD - NU,), jnp.int32)
    ids_all = jnp.concatenate([user_ids, idpad, item_ids, idpad])
    epad = EPAD - E
    srcfill = jnp.arange(epad, dtype=jnp.int32) % NPAD
    dstfill = NU + (jnp.arange(epad, dtype=jnp.int32) % (NPAD - NU))
    src_all = jnp.concatenate([
        edge_index_u2i[0], srcfill,
        edge_index_i2u[0] + NPAD, srcfill + NPAD,
    ])
    dst_all = jnp.concatenate([
        edge_index_u2i[1], dstfill,
        edge_index_i2u[1], dstfill,
    ])

    Zf, counts = _embed_hist(ids_all, user_table, item_table, dst_all)
    Z = Zf.reshape(2, NPAD, EMB)
    C4 = counts.reshape(2, NBLK, 1, BLK)

    b1u = b1_i2u.reshape(1, HID)
    b1i = b1_u2i.reshape(1, HID)
    b2u = b2_i2u.reshape(1, HID)
    b2i = b2_u2i.reshape(1, HID)
    Wp2 = Wp.reshape(2, HID)
    bp2 = jnp.broadcast_to(bp.reshape(1, 1), (1, HID))

    S1 = _segsum(Zf, src_all, dst_all).reshape(2, NPAD, EMB)
    H1 = _dense_layer(S1, C4, Z, Wn1_i2u, Ws1_i2u, b1u, Wn1_u2i, Ws1_u2i, b1i)
    S2 = _segsum(H1.reshape(2 * NPAD, HID), src_all, dst_all).reshape(2, NPAD, HID)
    out = _dense_final(S2, C4, H1, Wn2_i2u, Ws2_i2u, b2u,
                       Wn2_u2i, Ws2_u2i, b2i, Wp2, bp2)
    return out.reshape(NPAD, 1)[:NU]


# speedup vs baseline: 2.2734x; 1.0425x over previous
import jax
import jax.numpy as jnp
from jax import lax
from jax.experimental import pallas as pl
from jax.experimental.pallas import tpu as pltpu
from jax.experimental.pallas import tpu_sc as plsc

NU = 10000
NI = 10000
E = 320000
EMB = 128
HID = 128

NPAD = 10240
RPT = NPAD // 16
K = 128
CHUNKS = 158
EPT = CHUNKS * K
EPAD = EPT * 16
NBUF = 2
NBLK = 40
BLK = NPAD // NBLK
GPW = NPAD // 32
GK = 80


def _embed_hist_body(ids_ref, utab_ref, itab_ref, dst_ref,
                     z_ref, cnt_ref,
                     idx_v, gidx_v, grow_v, ones_v, zrow_v, cacc_sh, sem):
    c = lax.axis_index("c")
    s = lax.axis_index("s")
    wid = c * 16 + s
    zeros16 = jnp.zeros((16,), jnp.float32)
    ones16 = jnp.ones((16,), jnp.float32)

    def obody(i, _):
        ones_v[pl.ds(i * 16, 16)] = ones16
        return 0
    lax.fori_loop(0, K // 16, obody, 0)

    def zcbody(i, _):
        zrow_v[pl.ds(i * 16, 16)] = zeros16
        return 0
    lax.fori_loop(0, RPT // 16, zcbody, 0)
    pltpu.sync_copy(zrow_v, cacc_sh.at[pl.ds(s * RPT, RPT)])
    plsc.subcore_barrier()

    for j in range(GPW // GK):
        base = wid * GPW + j * GK
        pltpu.sync_copy(ids_ref.at[pl.ds(base, GK)], gidx_v)
        pltpu.async_copy(utab_ref.at[gidx_v], grow_v, sem).wait()
        pltpu.sync_copy(grow_v, z_ref.at[pl.ds(base, GK)])
    for j in range(GPW // GK):
        base = NPAD + wid * GPW + j * GK
        pltpu.sync_copy(ids_ref.at[pl.ds(base, GK)], gidx_v)
        pltpu.async_copy(itab_ref.at[gidx_v], grow_v, sem).wait()
        pltpu.sync_copy(grow_v, z_ref.at[pl.ds(base, GK)])

    def hbody(j, _):
        off = c * EPAD + s * EPT + j * K
        pltpu.sync_copy(dst_ref.at[pl.ds(off, K)], idx_v)
        pltpu.sync_copy(ones_v, cacc_sh.at[idx_v], add=True)
        return 0
    lax.fori_loop(0, CHUNKS, hbody, 0)

    plsc.subcore_barrier()
    pltpu.sync_copy(cacc_sh.at[pl.ds(s * RPT, RPT)],
                    cnt_ref.at[pl.ds(c * NPAD + s * RPT, RPT)])


def _segsum_body(t_ref, src_ref, dst_ref, s_out_ref,
                 isrc_v, idst_v, rows_v, isrc2_v, idst2_v, rows2_v,
                 acc_sh, sem, sem2, ssem, ssem2, xsem0, xsem1, xsem2, xsem3):
    c = lax.axis_index("c")
    s = lax.axis_index("s")
    ebase = c * EPAD + s * EPT
    zeros16 = jnp.zeros((16,), jnp.float32)

    def zbody(i, _):
        for k in range(EMB // 16):
            rows_v[i, pl.ds(k * 16, 16)] = zeros16
        return 0
    lax.fori_loop(0, K, zbody, 0)
    for j in range(RPT // K):
        pltpu.sync_copy(rows_v, acc_sh.at[pl.ds(s * RPT + j * K, K)])
    plsc.subcore_barrier()

    def body(pr, _):
        off0 = ebase + (2 * pr) * K
        off1 = off0 + K
        a0 = pltpu.async_copy(src_ref.at[pl.ds(off0, K)], isrc_v, xsem0)
        b0 = pltpu.async_copy(dst_ref.at[pl.ds(off0, K)], idst_v, xsem1)
        a1 = pltpu.async_copy(src_ref.at[pl.ds(off1, K)], isrc2_v, xsem2)
        b1 = pltpu.async_copy(dst_ref.at[pl.ds(off1, K)], idst2_v, xsem3)
        a0.wait()
        g0 = pltpu.async_copy(t_ref.at[isrc_v], rows_v, sem)
        a1.wait()
        g1 = pltpu.async_copy(t_ref.at[isrc2_v], rows2_v, sem2)
        g0.wait()
        b0.wait()
        s0 = pltpu.async_copy(rows_v, acc_sh.at[idst_v], ssem, add=True)
        g1.wait()
        b1.wait()
        s1 = pltpu.async_copy(rows2_v, acc_sh.at[idst2_v], ssem2, add=True)
        s0.wait()
        s1.wait()
        return 0
    lax.fori_loop(0, CHUNKS // 2, body, 0)

    plsc.subcore_barrier()
    pltpu.sync_copy(acc_sh.at[pl.ds(s * RPT, RPT)],
                    s_out_ref.at[pl.ds(c * NPAD + s * RPT, RPT)])


def _dense_layer_body(s_ref, c_ref, z_ref,
                      wnu_ref, wsu_ref, bu_ref, wni_ref, wsi_ref, bi_ref,
                      h_ref):
    cu = jnp.clip(c_ref[1, 0, 0, :], 1.0, None)
    ci = jnp.clip(c_ref[0, 0, 0, :], 1.0, None)
    agg_u = s_ref[1] / cu[:, None]
    agg_i = s_ref[0] / ci[:, None]
    nu = (jnp.dot(agg_u, wnu_ref[...], preferred_element_type=jnp.float32)
          + jnp.dot(z_ref[0], wsu_ref[...], preferred_element_type=jnp.float32)
          + bu_ref[0, :])
    ni = (jnp.dot(agg_i, wni_ref[...], preferred_element_type=jnp.float32)
          + jnp.dot(z_ref[1], wsi_ref[...], preferred_element_type=jnp.float32)
          + bi_ref[0, :])
    h_ref[0] = jnp.maximum(nu, 0.0)
    h_ref[1] = jnp.maximum(ni, 0.0)


def _dense_final_body(s_ref, c_ref, z_ref,
                      wnu_ref, wsu_ref, bu_ref, wni_ref, wsi_ref, bi_ref,
                      wp_ref, bp_ref,
                      out_ref):
    cu = jnp.clip(c_ref[1, 0, 0, :], 1.0, None)
    ci = jnp.clip(c_ref[0, 0, 0, :], 1.0, None)
    agg_u = s_ref[1] / cu[:, None]
    agg_i = s_ref[0] / ci[:, None]
    nu = (jnp.dot(agg_u, wnu_ref[...], preferred_element_type=jnp.float32)
          + jnp.dot(z_ref[0], wsu_ref[...], preferred_element_type=jnp.float32)
          + bu_ref[0, :])
    ni = (jnp.dot(agg_i, wni_ref[...], preferred_element_type=jnp.float32)
          + jnp.dot(z_ref[1], wsi_ref[...], preferred_element_type=jnp.float32)
          + bi_ref[0, :])
    hu = jnp.maximum(nu, 0.0)
    hi = jnp.maximum(ni, 0.0)
    res = (jnp.sum(hu * wp_ref[0, :][None, :], axis=1)
           + jnp.sum(hi * wp_ref[1, :][None, :], axis=1)
           + bp_ref[0, 0])
    out_ref[0, 0, :] = res


def _sc_mesh():
    return plsc.VectorSubcoreMesh(core_axis_name="c", subcore_axis_name="s")


def _embed_hist(ids_all, user_table, item_table, dst_all):
    return pl.kernel(
        _embed_hist_body,
        out_type=(
            jax.ShapeDtypeStruct((2 * NPAD, EMB), jnp.float32),
            jax.ShapeDtypeStruct((2 * NPAD,), jnp.float32),
        ),
        mesh=_sc_mesh(),
        scratch_types=[
            pltpu.VMEM((K,), jnp.int32),
            pltpu.VMEM((GK,), jnp.int32),
            pltpu.VMEM((GK, EMB), jnp.float32),
            pltpu.VMEM((K,), jnp.float32),
            pltpu.VMEM((RPT,), jnp.float32),
            pltpu.VMEM_SHARED((NPAD,), jnp.float32),
            pltpu.SemaphoreType.DMA,
        ],
    )(ids_all, user_table, item_table, dst_all)


def _segsum(table_flat, src_all, dst_all):
    return pl.kernel(
        _segsum_body,
        out_type=jax.ShapeDtypeStruct((2 * NPAD, EMB), jnp.float32),
        mesh=_sc_mesh(),
        scratch_types=[
            pltpu.VMEM((K,), jnp.int32),
            pltpu.VMEM((K,), jnp.int32),
            pltpu.VMEM((K, EMB), jnp.float32),
            pltpu.VMEM((K,), jnp.int32),
            pltpu.VMEM((K,), jnp.int32),
            pltpu.VMEM((K, EMB), jnp.float32),
            pltpu.VMEM_SHARED((NPAD, EMB), jnp.float32),
            pltpu.SemaphoreType.DMA,
            pltpu.SemaphoreType.DMA,
            pltpu.SemaphoreType.DMA,
            pltpu.SemaphoreType.DMA,
            pltpu.SemaphoreType.DMA,
            pltpu.SemaphoreType.DMA,
            pltpu.SemaphoreType.DMA,
            pltpu.SemaphoreType.DMA,
        ],
    )(table_flat, src_all, dst_all)


def _dense_layer(S, C4, Z, Wnu, Wsu, bu, Wni, Wsi, bi):
    wspec = pl.BlockSpec((EMB, HID), lambda i: (0, 0))
    bspec = pl.BlockSpec((1, HID), lambda i: (0, 0))
    return pl.pallas_call(
        _dense_layer_body,
        grid=(NBLK,),
        in_specs=[
            pl.BlockSpec((2, BLK, EMB), lambda i: (0, i, 0)),
            pl.BlockSpec((2, 1, 1, BLK), lambda i: (0, i, 0, 0)),
            pl.BlockSpec((2, BLK, EMB), lambda i: (0, i, 0)),
            wspec, wspec, bspec, wspec, wspec, bspec,
        ],
        out_specs=pl.BlockSpec((2, BLK, HID), lambda i: (0, i, 0)),
        out_shape=jax.ShapeDtypeStruct((2, NPAD, HID), jnp.float32),
    )(S, C4, Z, Wnu, Wsu, bu, Wni, Wsi, bi)


def _dense_final(S, C4, Z, Wnu, Wsu, bu, Wni, Wsi, bi, Wp2, bp2):
    wspec = pl.BlockSpec((HID, HID), lambda i: (0, 0))
    bspec = pl.BlockSpec((1, HID), lambda i: (0, 0))
    return pl.pallas_call(
        _dense_final_body,
        grid=(NBLK,),
        in_specs=[
            pl.BlockSpec((2, BLK, HID), lambda i: (0, i, 0)),
            pl.BlockSpec((2, 1, 1, BLK), lambda i: (0, i, 0, 0)),
            pl.BlockSpec((2, BLK, HID), lambda i: (0, i, 0)),
            wspec, wspec, bspec, wspec, wspec, bspec,
            pl.BlockSpec((2, HID), lambda i: (0, 0)),
            bspec,
        ],
        out_specs=pl.BlockSpec((1, 1, BLK), lambda i: (i, 0, 0)),
        out_shape=jax.ShapeDtypeStruct((NBLK, 1, BLK), jnp.float32),
    )(S, C4, Z, Wnu, Wsu, bu, Wni, Wsi, bi, Wp2, bp2)


def kernel(user_ids, item_ids, edge_index_u2i, edge_index_i2u,
           user_table, item_table,
           Wn1_u2i, Ws1_u2i, b1_u2i, Wn1_i2u, Ws1_i2u, b1_i2u,
           Wn2_u2i, Ws2_u2i, b2_u2i, Wn2_i2u, Ws2_i2u, b2_i2u,
           Wp, bp):
    idpad = jnp.zeros((NPAD - NU,), jnp.int32)
    ids_all = jnp.concatenate([user_ids, idpad, item_ids, idpad])
    epad = EPAD - E
    srcfill = jnp.arange(epad, dtype=jnp.int32) % NPAD
    dstfill = NU + (jnp.arange(epad, dtype=jnp.int32) % (NPAD - NU))
    src_all = jnp.concatenate([
        edge_index_u2i[0], srcfill,
        edge_index_i2u[0] + NPAD, srcfill + NPAD,
    ])
    dst_all = jnp.concatenate([
        edge_index_u2i[1], dstfill,
        edge_index_i2u[1], dstfill,
    ])

    Zf, counts = _embed_hist(ids_all, user_table, item_table, dst_all)
    Z = Zf.reshape(2, NPAD, EMB)
    C4 = counts.reshape(2, NBLK, 1, BLK)

    b1u = b1_i2u.reshape(1, HID)
    b1i = b1_u2i.reshape(1, HID)
    b2u = b2_i2u.reshape(1, HID)
    b2i = b2_u2i.reshape(1, HID)
    Wp2 = Wp.reshape(2, HID)
    bp2 = jnp.broadcast_to(bp.reshape(1, 1), (1, HID))

    S1 = _segsum(Zf, src_all, dst_all).reshape(2, NPAD, EMB)
    H1 = _dense_layer(S1, C4, Z, Wn1_i2u, Ws1_i2u, b1u, Wn1_u2i, Ws1_u2i, b1i)
    S2 = _segsum(H1.reshape(2 * NPAD, HID), src_all, dst_all).reshape(2, NPAD, HID)
    out = _dense_final(S2, C4, H1, Wn2_i2u, Ws2_i2u, b2u,
                       Wn2_u2i, Ws2_u2i, b2i, Wp2, bp2)
    return out.reshape(NPAD, 1)[:NU]


# speedup vs baseline: 2.3761x; 1.0452x over previous
import jax
import jax.numpy as jnp
from jax import lax
from jax.experimental import pallas as pl
from jax.experimental.pallas import tpu as pltpu
from jax.experimental.pallas import tpu_sc as plsc

NU = 10000
NI = 10000
E = 320000
EMB = 128
HID = 128

NPAD = 10240
RPT = NPAD // 16
K = 128
CHUNKS = 158
EPT = CHUNKS * K
EPAD = EPT * 16
NBUF = 2
NBLK = 40
BLK = NPAD // NBLK
GPW = NPAD // 32
GK = 80


def _embed_hist_body(ids_ref, utab_ref, itab_ref, dst_ref,
                     z_ref, cnt_ref,
                     idx_v, idx2_v, gidx_v, grow_v, ones_v, zrow_v, cacc_sh,
                     sem, hx0, hx1, hs0, hs1):
    c = lax.axis_index("c")
    s = lax.axis_index("s")
    wid = c * 16 + s
    zeros16 = jnp.zeros((16,), jnp.float32)
    ones16 = jnp.ones((16,), jnp.float32)

    def obody(i, _):
        ones_v[pl.ds(i * 16, 16)] = ones16
        return 0
    lax.fori_loop(0, K // 16, obody, 0)

    def zcbody(i, _):
        zrow_v[pl.ds(i * 16, 16)] = zeros16
        return 0
    lax.fori_loop(0, RPT // 16, zcbody, 0)
    pltpu.sync_copy(zrow_v, cacc_sh.at[pl.ds(s * RPT, RPT)])
    plsc.subcore_barrier()

    for j in range(GPW // GK):
        base = wid * GPW + j * GK
        pltpu.sync_copy(ids_ref.at[pl.ds(base, GK)], gidx_v)
        pltpu.async_copy(utab_ref.at[gidx_v], grow_v, sem).wait()
        pltpu.sync_copy(grow_v, z_ref.at[pl.ds(base, GK)])
    for j in range(GPW // GK):
        base = NPAD + wid * GPW + j * GK
        pltpu.sync_copy(ids_ref.at[pl.ds(base, GK)], gidx_v)
        pltpu.async_copy(itab_ref.at[gidx_v], grow_v, sem).wait()
        pltpu.sync_copy(grow_v, z_ref.at[pl.ds(base, GK)])

    def hbody(pr, _):
        off0 = c * EPAD + s * EPT + (2 * pr) * K
        off1 = off0 + K
        a0 = pltpu.async_copy(dst_ref.at[pl.ds(off0, K)], idx_v, hx0)
        a1 = pltpu.async_copy(dst_ref.at[pl.ds(off1, K)], idx2_v, hx1)
        a0.wait()
        h0 = pltpu.async_copy(ones_v, cacc_sh.at[idx_v], hs0, add=True)
        a1.wait()
        h1 = pltpu.async_copy(ones_v, cacc_sh.at[idx2_v], hs1, add=True)
        h0.wait()
        h1.wait()
        return 0
    lax.fori_loop(0, CHUNKS // 2, hbody, 0)

    plsc.subcore_barrier()
    pltpu.sync_copy(cacc_sh.at[pl.ds(s * RPT, RPT)],
                    cnt_ref.at[pl.ds(c * NPAD + s * RPT, RPT)])


def _segsum_body(t_ref, src_ref, dst_ref, s_out_ref,
                 isrc_v, idst_v, rows_v, isrc2_v, idst2_v, rows2_v,
                 acc_sh, sem, sem2, ssem, ssem2, xsem0, xsem1, xsem2, xsem3):
    c = lax.axis_index("c")
    s = lax.axis_index("s")
    ebase = c * EPAD + s * EPT
    zeros16 = jnp.zeros((16,), jnp.float32)

    def zbody(i, _):
        for k in range(EMB // 16):
            rows_v[i, pl.ds(k * 16, 16)] = zeros16
        return 0
    lax.fori_loop(0, K, zbody, 0)
    for j in range(RPT // K):
        pltpu.sync_copy(rows_v, acc_sh.at[pl.ds(s * RPT + j * K, K)])
    plsc.subcore_barrier()

    def body(pr, _):
        off0 = ebase + (2 * pr) * K
        off1 = off0 + K
        a0 = pltpu.async_copy(src_ref.at[pl.ds(off0, K)], isrc_v, xsem0)
        b0 = pltpu.async_copy(dst_ref.at[pl.ds(off0, K)], idst_v, xsem1)
        a1 = pltpu.async_copy(src_ref.at[pl.ds(off1, K)], isrc2_v, xsem2)
        b1 = pltpu.async_copy(dst_ref.at[pl.ds(off1, K)], idst2_v, xsem3)
        a0.wait()
        g0 = pltpu.async_copy(t_ref.at[isrc_v], rows_v, sem)
        a1.wait()
        g1 = pltpu.async_copy(t_ref.at[isrc2_v], rows2_v, sem2)
        g0.wait()
        b0.wait()
        s0 = pltpu.async_copy(rows_v, acc_sh.at[idst_v], ssem, add=True)
        g1.wait()
        b1.wait()
        s1 = pltpu.async_copy(rows2_v, acc_sh.at[idst2_v], ssem2, add=True)
        s0.wait()
        s1.wait()
        return 0
    lax.fori_loop(0, CHUNKS // 2, body, 0)

    plsc.subcore_barrier()
    pltpu.sync_copy(acc_sh.at[pl.ds(s * RPT, RPT)],
                    s_out_ref.at[pl.ds(c * NPAD + s * RPT, RPT)])


def _dense_layer_body(s_ref, c_ref, z_ref,
                      wnu_ref, wsu_ref, bu_ref, wni_ref, wsi_ref, bi_ref,
                      h_ref):
    cu = jnp.clip(c_ref[1, 0, 0, :], 1.0, None)
    ci = jnp.clip(c_ref[0, 0, 0, :], 1.0, None)
    agg_u = s_ref[1] / cu[:, None]
    agg_i = s_ref[0] / ci[:, None]
    nu = (jnp.dot(agg_u, wnu_ref[...], preferred_element_type=jnp.float32)
          + jnp.dot(z_ref[0], wsu_ref[...], preferred_element_type=jnp.float32)
          + bu_ref[0, :])
    ni = (jnp.dot(agg_i, wni_ref[...], preferred_element_type=jnp.float32)
          + jnp.dot(z_ref[1], wsi_ref[...], preferred_element_type=jnp.float32)
          + bi_ref[0, :])
    h_ref[0] = jnp.maximum(nu, 0.0)
    h_ref[1] = jnp.maximum(ni, 0.0)


def _dense_final_body(s_ref, c_ref, z_ref,
                      wnu_ref, wsu_ref, bu_ref, wni_ref, wsi_ref, bi_ref,
                      wp_ref, bp_ref,
                      out_ref):
    cu = jnp.clip(c_ref[1, 0, 0, :], 1.0, None)
    ci = jnp.clip(c_ref[0, 0, 0, :], 1.0, None)
    agg_u = s_ref[1] / cu[:, None]
    agg_i = s_ref[0] / ci[:, None]
    nu = (jnp.dot(agg_u, wnu_ref[...], preferred_element_type=jnp.float32)
          + jnp.dot(z_ref[0], wsu_ref[...], preferred_element_type=jnp.float32)
          + bu_ref[0, :])
    ni = (jnp.dot(agg_i, wni_ref[...], preferred_element_type=jnp.float32)
          + jnp.dot(z_ref[1], wsi_ref[...], preferred_element_type=jnp.float32)
          + bi_ref[0, :])
    hu = jnp.maximum(nu, 0.0)
    hi = jnp.maximum(ni, 0.0)
    res = (jnp.sum(hu * wp_ref[0, :][None, :], axis=1)
           + jnp.sum(hi * wp_ref[1, :][None, :], axis=1)
           + bp_ref[0, 0])
    out_ref[0, 0, :] = res


def _sc_mesh():
    return plsc.VectorSubcoreMesh(core_axis_name="c", subcore_axis_name="s")


def _embed_hist(ids_all, user_table, item_table, dst_all):
    return pl.kernel(
        _embed_hist_body,
        out_type=(
            jax.ShapeDtypeStruct((2 * NPAD, EMB), jnp.float32),
            jax.ShapeDtypeStruct((2 * NPAD,), jnp.float32),
        ),
        mesh=_sc_mesh(),
        scratch_types=[
            pltpu.VMEM((K,), jnp.int32),
            pltpu.VMEM((K,), jnp.int32),
            pltpu.VMEM((GK,), jnp.int32),
            pltpu.VMEM((GK, EMB), jnp.float32),
            pltpu.VMEM((K,), jnp.float32),
            pltpu.VMEM((RPT,), jnp.float32),
            pltpu.VMEM_SHARED((NPAD,), jnp.float32),
            pltpu.SemaphoreType.DMA,
            pltpu.SemaphoreType.DMA,
            pltpu.SemaphoreType.DMA,
            pltpu.SemaphoreType.DMA,
            pltpu.SemaphoreType.DMA,
        ],
    )(ids_all, user_table, item_table, dst_all)


def _segsum(table_flat, src_all, dst_all):
    return pl.kernel(
        _segsum_body,
        out_type=jax.ShapeDtypeStruct((2 * NPAD, EMB), jnp.float32),
        mesh=_sc_mesh(),
        scratch_types=[
            pltpu.VMEM((K,), jnp.int32),
            pltpu.VMEM((K,), jnp.int32),
            pltpu.VMEM((K, EMB), jnp.float32),
            pltpu.VMEM((K,), jnp.int32),
            pltpu.VMEM((K,), jnp.int32),
            pltpu.VMEM((K, EMB), jnp.float32),
            pltpu.VMEM_SHARED((NPAD, EMB), jnp.float32),
            pltpu.SemaphoreType.DMA,
            pltpu.SemaphoreType.DMA,
            pltpu.SemaphoreType.DMA,
            pltpu.SemaphoreType.DMA,
            pltpu.SemaphoreType.DMA,
            pltpu.SemaphoreType.DMA,
            pltpu.SemaphoreType.DMA,
            pltpu.SemaphoreType.DMA,
        ],
    )(table_flat, src_all, dst_all)


def _dense_layer(S, C4, Z, Wnu, Wsu, bu, Wni, Wsi, bi):
    wspec = pl.BlockSpec((EMB, HID), lambda i: (0, 0))
    bspec = pl.BlockSpec((1, HID), lambda i: (0, 0))
    return pl.pallas_call(
        _dense_layer_body,
        grid=(NBLK,),
        in_specs=[
            pl.BlockSpec((2, BLK, EMB), lambda i: (0, i, 0)),
            pl.BlockSpec((2, 1, 1, BLK), lambda i: (0, i, 0, 0)),
            pl.BlockSpec((2, BLK, EMB), lambda i: (0, i, 0)),
            wspec, wspec, bspec, wspec, wspec, bspec,
        ],
        out_specs=pl.BlockSpec((2, BLK, HID), lambda i: (0, i, 0)),
        out_shape=jax.ShapeDtypeStruct((2, NPAD, HID), jnp.float32),
    )(S, C4, Z, Wnu, Wsu, bu, Wni, Wsi, bi)


def _dense_final(S, C4, Z, Wnu, Wsu, bu, Wni, Wsi, bi, Wp2, bp2):
    wspec = pl.BlockSpec((HID, HID), lambda i: (0, 0))
    bspec = pl.BlockSpec((1, HID), lambda i: (0, 0))
    return pl.pallas_call(
        _dense_final_body,
        grid=(NBLK,),
        in_specs=[
            pl.BlockSpec((2, BLK, HID), lambda i: (0, i, 0)),
            pl.BlockSpec((2, 1, 1, BLK), lambda i: (0, i, 0, 0)),
            pl.BlockSpec((2, BLK, HID), lambda i: (0, i, 0)),
            wspec, wspec, bspec, wspec, wspec, bspec,
            pl.BlockSpec((2, HID), lambda i: (0, 0)),
            bspec,
        ],
        out_specs=pl.BlockSpec((1, 1, BLK), lambda i: (i, 0, 0)),
        out_shape=jax.ShapeDtypeStruct((NBLK, 1, BLK), jnp.float32),
    )(S, C4, Z, Wnu, Wsu, bu, Wni, Wsi, bi, Wp2, bp2)


def kernel(user_ids, item_ids, edge_index_u2i, edge_index_i2u,
           user_table, item_table,
           Wn1_u2i, Ws1_u2i, b1_u2i, Wn1_i2u, Ws1_i2u, b1_i2u,
           Wn2_u2i, Ws2_u2i, b2_u2i, Wn2_i2u, Ws2_i2u, b2_i2u,
           Wp, bp):
    idpad = jnp.zeros((NPAD - NU,), jnp.int32)
    ids_all = jnp.concatenate([user_ids, idpad, item_ids, idpad])
    epad = EPAD - E
    srcfill = jnp.arange(epad, dtype=jnp.int32) % NPAD
    dstfill = NU + (jnp.arange(epad, dtype=jnp.int32) % (NPAD - NU))
    src_all = jnp.concatenate([
        edge_index_u2i[0], srcfill,
        edge_index_i2u[0] + NPAD, srcfill + NPAD,
    ])
    dst_all = jnp.concatenate([
        edge_index_u2i[1], dstfill,
        edge_index_i2u[1], dstfill,
    ])

    Zf, counts = _embed_hist(ids_all, user_table, item_table, dst_all)
    Z = Zf.reshape(2, NPAD, EMB)
    C4 = counts.reshape(2, NBLK, 1, BLK)

    b1u = b1_i2u.reshape(1, HID)
    b1i = b1_u2i.reshape(1, HID)
    b2u = b2_i2u.reshape(1, HID)
    b2i = b2_u2i.reshape(1, HID)
    Wp2 = Wp.reshape(2, HID)
    bp2 = jnp.broadcast_to(bp.reshape(1, 1), (1, HID))

    S1 = _segsum(Zf, src_all, dst_all).reshape(2, NPAD, EMB)
    H1 = _dense_layer(S1, C4, Z, Wn1_i2u, Ws1_i2u, b1u, Wn1_u2i, Ws1_u2i, b1i)
    S2 = _segsum(H1.reshape(2 * NPAD, HID), src_all, dst_all).reshape(2, NPAD, HID)
    out = _dense_final(S2, C4, H1, Wn2_i2u, Ws2_i2u, b2u,
                       Wn2_u2i, Ws2_u2i, b2i, Wp2, bp2)
    return out.reshape(NPAD, 1)[:NU]


# speedup vs baseline: 2.4813x; 1.0443x over previous
import jax
import jax.numpy as jnp
from jax import lax
from jax.experimental import pallas as pl
from jax.experimental.pallas import tpu as pltpu
from jax.experimental.pallas import tpu_sc as plsc

NU = 10000
NI = 10000
E = 320000
EMB = 128
HID = 128

NPAD = 10240
RPT = NPAD // 16
K = 128
CHUNKS = 160
EPT = CHUNKS * K
EPAD = EPT * 16
NBUF = 2
NBLK = 40
BLK = NPAD // NBLK
GPW = NPAD // 32
GK = 80


def _embed_hist_body(ids_ref, utab_ref, itab_ref, dst_ref,
                     z_ref, cnt_ref,
                     idx_v, idx2_v, gidx_v, grow_v, ones_v, zrow_v, cacc_sh,
                     sem, hx0, hx1, hs0, hs1):
    c = lax.axis_index("c")
    s = lax.axis_index("s")
    wid = c * 16 + s
    zeros16 = jnp.zeros((16,), jnp.float32)
    ones16 = jnp.ones((16,), jnp.float32)

    def obody(i, _):
        ones_v[pl.ds(i * 16, 16)] = ones16
        return 0
    lax.fori_loop(0, K // 16, obody, 0)

    def zcbody(i, _):
        zrow_v[pl.ds(i * 16, 16)] = zeros16
        return 0
    lax.fori_loop(0, RPT // 16, zcbody, 0)
    pltpu.sync_copy(zrow_v, cacc_sh.at[pl.ds(s * RPT, RPT)])
    plsc.subcore_barrier()

    for j in range(GPW // GK):
        base = wid * GPW + j * GK
        pltpu.sync_copy(ids_ref.at[pl.ds(base, GK)], gidx_v)
        pltpu.async_copy(utab_ref.at[gidx_v], grow_v, sem).wait()
        pltpu.sync_copy(grow_v, z_ref.at[pl.ds(base, GK)])
    for j in range(GPW // GK):
        base = NPAD + wid * GPW + j * GK
        pltpu.sync_copy(ids_ref.at[pl.ds(base, GK)], gidx_v)
        pltpu.async_copy(itab_ref.at[gidx_v], grow_v, sem).wait()
        pltpu.sync_copy(grow_v, z_ref.at[pl.ds(base, GK)])

    def hbody(pr, _):
        off0 = c * EPAD + s * EPT + (2 * pr) * K
        off1 = off0 + K
        a0 = pltpu.async_copy(dst_ref.at[pl.ds(off0, K)], idx_v, hx0)
        a1 = pltpu.async_copy(dst_ref.at[pl.ds(off1, K)], idx2_v, hx1)
        a0.wait()
        h0 = pltpu.async_copy(ones_v, cacc_sh.at[idx_v], hs0, add=True)
        a1.wait()
        h1 = pltpu.async_copy(ones_v, cacc_sh.at[idx2_v], hs1, add=True)
        h0.wait()
        h1.wait()
        return 0
    lax.fori_loop(0, CHUNKS // 2, hbody, 0)

    plsc.subcore_barrier()
    pltpu.sync_copy(cacc_sh.at[pl.ds(s * RPT, RPT)],
                    cnt_ref.at[pl.ds(c * NPAD + s * RPT, RPT)])


def _segsum_body(t_ref, src_ref, dst_ref, s_out_ref,
                 isrc_v, idst_v, rows_v, isrc2_v, idst2_v, rows2_v,
                 isrc3_v, idst3_v, isrc4_v, idst4_v,
                 acc_sh, sem, sem2, ssem, ssem2, xsem0, xsem1, xsem2, xsem3):
    c = lax.axis_index("c")
    s = lax.axis_index("s")
    ebase = c * EPAD + s * EPT
    zeros16 = jnp.zeros((16,), jnp.float32)

    def zbody(i, _):
        for k in range(EMB // 16):
            rows_v[i, pl.ds(k * 16, 16)] = zeros16
        return 0
    lax.fori_loop(0, K, zbody, 0)
    for j in range(RPT // K):
        pltpu.sync_copy(rows_v, acc_sh.at[pl.ds(s * RPT + j * K, K)])
    plsc.subcore_barrier()

    def body(pr, _):
        off0 = ebase + (4 * pr) * K
        off1 = off0 + K
        off2 = off0 + 2 * K
        off3 = off0 + 3 * K
        a0 = pltpu.async_copy(src_ref.at[pl.ds(off0, K)], isrc_v, xsem0)
        b0 = pltpu.async_copy(dst_ref.at[pl.ds(off0, K)], idst_v, xsem1)
        a1 = pltpu.async_copy(src_ref.at[pl.ds(off1, K)], isrc2_v, xsem2)
        b1 = pltpu.async_copy(dst_ref.at[pl.ds(off1, K)], idst2_v, xsem3)
        a2 = pltpu.async_copy(src_ref.at[pl.ds(off2, K)], isrc3_v, xsem0)
        b2 = pltpu.async_copy(dst_ref.at[pl.ds(off2, K)], idst3_v, xsem1)
        a3 = pltpu.async_copy(src_ref.at[pl.ds(off3, K)], isrc4_v, xsem2)
        b3 = pltpu.async_copy(dst_ref.at[pl.ds(off3, K)], idst4_v, xsem3)
        a0.wait()
        g0 = pltpu.async_copy(t_ref.at[isrc_v], rows_v, sem)
        a1.wait()
        g1 = pltpu.async_copy(t_ref.at[isrc2_v], rows2_v, sem2)
        g0.wait()
        b0.wait()
        s0 = pltpu.async_copy(rows_v, acc_sh.at[idst_v], ssem, add=True)
        g1.wait()
        b1.wait()
        s1 = pltpu.async_copy(rows2_v, acc_sh.at[idst2_v], ssem2, add=True)
        s0.wait()
        a2.wait()
        g2 = pltpu.async_copy(t_ref.at[isrc3_v], rows_v, sem)
        s1.wait()
        a3.wait()
        g3 = pltpu.async_copy(t_ref.at[isrc4_v], rows2_v, sem2)
        g2.wait()
        b2.wait()
        s2 = pltpu.async_copy(rows_v, acc_sh.at[idst3_v], ssem, add=True)
        g3.wait()
        b3.wait()
        s3 = pltpu.async_copy(rows2_v, acc_sh.at[idst4_v], ssem2, add=True)
        s2.wait()
        s3.wait()
        return 0
    lax.fori_loop(0, CHUNKS // 4, body, 0)

    plsc.subcore_barrier()
    pltpu.sync_copy(acc_sh.at[pl.ds(s * RPT, RPT)],
                    s_out_ref.at[pl.ds(c * NPAD + s * RPT, RPT)])


def _dense_layer_body(s_ref, c_ref, z_ref,
                      wnu_ref, wsu_ref, bu_ref, wni_ref, wsi_ref, bi_ref,
                      h_ref):
    cu = jnp.clip(c_ref[1, 0, 0, :], 1.0, None)
    ci = jnp.clip(c_ref[0, 0, 0, :], 1.0, None)
    agg_u = s_ref[1] / cu[:, None]
    agg_i = s_ref[0] / ci[:, None]
    nu = (jnp.dot(agg_u, wnu_ref[...], preferred_element_type=jnp.float32)
          + jnp.dot(z_ref[0], wsu_ref[...], preferred_element_type=jnp.float32)
          + bu_ref[0, :])
    ni = (jnp.dot(agg_i, wni_ref[...], preferred_element_type=jnp.float32)
          + jnp.dot(z_ref[1], wsi_ref[...], preferred_element_type=jnp.float32)
          + bi_ref[0, :])
    h_ref[0] = jnp.maximum(nu, 0.0)
    h_ref[1] = jnp.maximum(ni, 0.0)


def _dense_final_body(s_ref, c_ref, z_ref,
                      wnu_ref, wsu_ref, bu_ref, wni_ref, wsi_ref, bi_ref,
                      wp_ref, bp_ref,
                      out_ref):
    cu = jnp.clip(c_ref[1, 0, 0, :], 1.0, None)
    ci = jnp.clip(c_ref[0, 0, 0, :], 1.0, None)
    agg_u = s_ref[1] / cu[:, None]
    agg_i = s_ref[0] / ci[:, None]
    nu = (jnp.dot(agg_u, wnu_ref[...], preferred_element_type=jnp.float32)
          + jnp.dot(z_ref[0], wsu_ref[...], preferred_element_type=jnp.float32)
          + bu_ref[0, :])
    ni = (jnp.dot(agg_i, wni_ref[...], preferred_element_type=jnp.float32)
          + jnp.dot(z_ref[1], wsi_ref[...], preferred_element_type=jnp.float32)
          + bi_ref[0, :])
    hu = jnp.maximum(nu, 0.0)
    hi = jnp.maximum(ni, 0.0)
    res = (jnp.sum(hu * wp_ref[0, :][None, :], axis=1)
           + jnp.sum(hi * wp_ref[1, :][None, :], axis=1)
           + bp_ref[0, 0])
    out_ref[0, 0, :] = res


def _sc_mesh():
    return plsc.VectorSubcoreMesh(core_axis_name="c", subcore_axis_name="s")


def _embed_hist(ids_all, user_table, item_table, dst_all):
    return pl.kernel(
        _embed_hist_body,
        out_type=(
            jax.ShapeDtypeStruct((2 * NPAD, EMB), jnp.float32),
            jax.ShapeDtypeStruct((2 * NPAD,), jnp.float32),
        ),
        mesh=_sc_mesh(),
        scratch_types=[
            pltpu.VMEM((K,), jnp.int32),
            pltpu.VMEM((K,), jnp.int32),
            pltpu.VMEM((GK,), jnp.int32),
            pltpu.VMEM((GK, EMB), jnp.float32),
            pltpu.VMEM((K,), jnp.float32),
            pltpu.VMEM((RPT,), jnp.float32),
            pltpu.VMEM_SHARED((NPAD,), jnp.float32),
            pltpu.SemaphoreType.DMA,
            pltpu.SemaphoreType.DMA,
            pltpu.SemaphoreType.DMA,
            pltpu.SemaphoreType.DMA,
            pltpu.SemaphoreType.DMA,
        ],
    )(ids_all, user_table, item_table, dst_all)


def _segsum(table_flat, src_all, dst_all):
    return pl.kernel(
        _segsum_body,
        out_type=jax.ShapeDtypeStruct((2 * NPAD, EMB), jnp.float32),
        mesh=_sc_mesh(),
        scratch_types=[
            pltpu.VMEM((K,), jnp.int32),
            pltpu.VMEM((K,), jnp.int32),
            pltpu.VMEM((K, EMB), jnp.float32),
            pltpu.VMEM((K,), jnp.int32),
            pltpu.VMEM((K,), jnp.int32),
            pltpu.VMEM((K, EMB), jnp.float32),
            pltpu.VMEM((K,), jnp.int32),
            pltpu.VMEM((K,), jnp.int32),
            pltpu.VMEM((K,), jnp.int32),
            pltpu.VMEM((K,), jnp.int32),
            pltpu.VMEM_SHARED((NPAD, EMB), jnp.float32),
            pltpu.SemaphoreType.DMA,
            pltpu.SemaphoreType.DMA,
            pltpu.SemaphoreType.DMA,
            pltpu.SemaphoreType.DMA,
            pltpu.SemaphoreType.DMA,
            pltpu.SemaphoreType.DMA,
            pltpu.SemaphoreType.DMA,
            pltpu.SemaphoreType.DMA,
        ],
    )(table_flat, src_all, dst_all)


def _dense_layer(S, C4, Z, Wnu, Wsu, bu, Wni, Wsi, bi):
    wspec = pl.BlockSpec((EMB, HID), lambda i: (0, 0))
    bspec = pl.BlockSpec((1, HID), lambda i: (0, 0))
    return pl.pallas_call(
        _dense_layer_body,
        grid=(NBLK,),
        in_specs=[
            pl.BlockSpec((2, BLK, EMB), lambda i: (0, i, 0)),
            pl.BlockSpec((2, 1, 1, BLK), lambda i: (0, i, 0, 0)),
            pl.BlockSpec((2, BLK, EMB), lambda i: (0, i, 0)),
            wspec, wspec, bspec, wspec, wspec, bspec,
        ],
        out_specs=pl.BlockSpec((2, BLK, HID), lambda i: (0, i, 0)),
        out_shape=jax.ShapeDtypeStruct((2, NPAD, HID), jnp.float32),
    )(S, C4, Z, Wnu, Wsu, bu, Wni, Wsi, bi)


def _dense_final(S, C4, Z, Wnu, Wsu, bu, Wni, Wsi, bi, Wp2, bp2):
    wspec = pl.BlockSpec((HID, HID), lambda i: (0, 0))
    bspec = pl.BlockSpec((1, HID), lambda i: (0, 0))
    return pl.pallas_call(
        _dense_final_body,
        grid=(NBLK,),
        in_specs=[
            pl.BlockSpec((2, BLK, HID), lambda i: (0, i, 0)),
            pl.BlockSpec((2, 1, 1, BLK), lambda i: (0, i, 0, 0)),
            pl.BlockSpec((2, BLK, HID), lambda i: (0, i, 0)),
            wspec, wspec, bspec, wspec, wspec, bspec,
            pl.BlockSpec((2, HID), lambda i: (0, 0)),
            bspec,
        ],
        out_specs=pl.BlockSpec((1, 1, BLK), lambda i: (i, 0, 0)),
        out_shape=jax.ShapeDtypeStruct((NBLK, 1, BLK), jnp.float32),
    )(S, C4, Z, Wnu, Wsu, bu, Wni, Wsi, bi, Wp2, bp2)


def kernel(user_ids, item_ids, edge_index_u2i, edge_index_i2u,
           user_table, item_table,
           Wn1_u2i, Ws1_u2i, b1_u2i, Wn1_i2u, Ws1_i2u, b1_i2u,
           Wn2_u2i, Ws2_u2i, b2_u2i, Wn2_i2u, Ws2_i2u, b2_i2u,
           Wp, bp):
    idpad = jnp.zeros((NPAD - NU,), jnp.int32)
    ids_all = jnp.concatenate([user_ids, idpad, item_ids, idpad])
    epad = EPAD - E
    srcfill = jnp.arange(epad, dtype=jnp.int32) % NPAD
    dstfill = NU + (jnp.arange(epad, dtype=jnp.int32) % (NPAD - NU))
    src_all = jnp.concatenate([
        edge_index_u2i[0], srcfill,
        edge_index_i2u[0] + NPAD, srcfill + NPAD,
    ])
    dst_all = jnp.concatenate([
        edge_index_u2i[1], dstfill,
        edge_index_i2u[1], dstfill,
    ])

    Zf, counts = _embed_hist(ids_all, user_table, item_table, dst_all)
    Z = Zf.reshape(2, NPAD, EMB)
    C4 = counts.reshape(2, NBLK, 1, BLK)

    b1u = b1_i2u.reshape(1, HID)
    b1i = b1_u2i.reshape(1, HID)
    b2u = b2_i2u.reshape(1, HID)
    b2i = b2_u2i.reshape(1, HID)
    Wp2 = Wp.reshape(2, HID)
    bp2 = jnp.broadcast_to(bp.reshape(1, 1), (1, HID))

    S1 = _segsum(Zf, src_all, dst_all).reshape(2, NPAD, EMB)
    H1 = _dense_layer(S1, C4, Z, Wn1_i2u, Ws1_i2u, b1u, Wn1_u2i, Ws1_u2i, b1i)
    S2 = _segsum(H1.reshape(2 * NPAD, HID), src_all, dst_all).reshape(2, NPAD, HID)
    out = _dense_final(S2, C4, H1, Wn2_i2u, Ws2_i2u, b2u,
                       Wn2_u2i, Ws2_u2i, b2i, Wp2, bp2)
    return out.reshape(NPAD, 1)[:NU]


# speedup vs baseline: 2.5407x; 1.0239x over previous
import jax
import jax.numpy as jnp
from jax import lax
from jax.experimental import pallas as pl
from jax.experimental.pallas import tpu as pltpu
from jax.experimental.pallas import tpu_sc as plsc

NU = 10000
NI = 10000
E = 320000
EMB = 128
HID = 128

NPAD = 10240
RPT = NPAD // 16
K = 128
CHUNKS = 160
EPT = CHUNKS * K
EPAD = EPT * 16
NBUF = 2
NBLK = 40
BLK = NPAD // NBLK
GPW = NPAD // 32
GK = 80


def _embed_hist_body(ids_ref, utab_ref, itab_ref, dst_ref,
                     z_ref, cnt_ref,
                     idx_v, idx2_v, idx3_v, idx4_v, gidx_v, grow_v, ones_v,
                     zrow_v, cacc_sh, sem, hx0, hx1, hs0, hs1):
    c = lax.axis_index("c")
    s = lax.axis_index("s")
    wid = c * 16 + s
    zeros16 = jnp.zeros((16,), jnp.float32)
    ones16 = jnp.ones((16,), jnp.float32)

    def obody(i, _):
        ones_v[pl.ds(i * 16, 16)] = ones16
        return 0
    lax.fori_loop(0, K // 16, obody, 0)

    def zcbody(i, _):
        zrow_v[pl.ds(i * 16, 16)] = zeros16
        return 0
    lax.fori_loop(0, RPT // 16, zcbody, 0)
    pltpu.sync_copy(zrow_v, cacc_sh.at[pl.ds(s * RPT, RPT)])
    plsc.subcore_barrier()

    for j in range(GPW // GK):
        base = wid * GPW + j * GK
        pltpu.sync_copy(ids_ref.at[pl.ds(base, GK)], gidx_v)
        pltpu.async_copy(utab_ref.at[gidx_v], grow_v, sem).wait()
        pltpu.sync_copy(grow_v, z_ref.at[pl.ds(base, GK)])
    for j in range(GPW // GK):
        base = NPAD + wid * GPW + j * GK
        pltpu.sync_copy(ids_ref.at[pl.ds(base, GK)], gidx_v)
        pltpu.async_copy(itab_ref.at[gidx_v], grow_v, sem).wait()
        pltpu.sync_copy(grow_v, z_ref.at[pl.ds(base, GK)])

    def hbody(pr, _):
        off0 = c * EPAD + s * EPT + (4 * pr) * K
        a0 = pltpu.async_copy(dst_ref.at[pl.ds(off0, K)], idx_v, hx0)
        a1 = pltpu.async_copy(dst_ref.at[pl.ds(off0 + K, K)], idx2_v, hx1)
        a2 = pltpu.async_copy(dst_ref.at[pl.ds(off0 + 2 * K, K)], idx3_v,
                              hx0)
        a3 = pltpu.async_copy(dst_ref.at[pl.ds(off0 + 3 * K, K)], idx4_v,
                              hx1)
        a0.wait()
        h0 = pltpu.async_copy(ones_v, cacc_sh.at[idx_v], hs0, add=True)
        a1.wait()
        h1 = pltpu.async_copy(ones_v, cacc_sh.at[idx2_v], hs1, add=True)
        h0.wait()
        a2.wait()
        h2 = pltpu.async_copy(ones_v, cacc_sh.at[idx3_v], hs0, add=True)
        h1.wait()
        a3.wait()
        h3 = pltpu.async_copy(ones_v, cacc_sh.at[idx4_v], hs1, add=True)
        h2.wait()
        h3.wait()
        return 0
    lax.fori_loop(0, CHUNKS // 4, hbody, 0)

    plsc.subcore_barrier()
    pltpu.sync_copy(cacc_sh.at[pl.ds(s * RPT, RPT)],
                    cnt_ref.at[pl.ds(c * NPAD + s * RPT, RPT)])


def _segsum_body(t_ref, src_ref, dst_ref, s_out_ref,
                 isrc_v, idst_v, rows_v, isrc2_v, idst2_v, rows2_v,
                 isrc3_v, idst3_v, isrc4_v, idst4_v,
                 acc_sh, sem, sem2, ssem, ssem2, xsem0, xsem1, xsem2, xsem3):
    c = lax.axis_index("c")
    s = lax.axis_index("s")
    ebase = c * EPAD + s * EPT
    zeros16 = jnp.zeros((16,), jnp.float32)

    def zbody(i, _):
        for k in range(EMB // 16):
            rows_v[i, pl.ds(k * 16, 16)] = zeros16
        return 0
    lax.fori_loop(0, K, zbody, 0)
    for j in range(RPT // K):
        pltpu.sync_copy(rows_v, acc_sh.at[pl.ds(s * RPT + j * K, K)])
    plsc.subcore_barrier()

    def body(pr, _):
        off0 = ebase + (4 * pr) * K
        off1 = off0 + K
        off2 = off0 + 2 * K
        off3 = off0 + 3 * K
        a0 = pltpu.async_copy(src_ref.at[pl.ds(off0, K)], isrc_v, xsem0)
        b0 = pltpu.async_copy(dst_ref.at[pl.ds(off0, K)], idst_v, xsem1)
        a1 = pltpu.async_copy(src_ref.at[pl.ds(off1, K)], isrc2_v, xsem2)
        b1 = pltpu.async_copy(dst_ref.at[pl.ds(off1, K)], idst2_v, xsem3)
        a2 = pltpu.async_copy(src_ref.at[pl.ds(off2, K)], isrc3_v, xsem0)
        b2 = pltpu.async_copy(dst_ref.at[pl.ds(off2, K)], idst3_v, xsem1)
        a3 = pltpu.async_copy(src_ref.at[pl.ds(off3, K)], isrc4_v, xsem2)
        b3 = pltpu.async_copy(dst_ref.at[pl.ds(off3, K)], idst4_v, xsem3)
        a0.wait()
        g0 = pltpu.async_copy(t_ref.at[isrc_v], rows_v, sem)
        a1.wait()
        g1 = pltpu.async_copy(t_ref.at[isrc2_v], rows2_v, sem2)
        g0.wait()
        b0.wait()
        s0 = pltpu.async_copy(rows_v, acc_sh.at[idst_v], ssem, add=True)
        g1.wait()
        b1.wait()
        s1 = pltpu.async_copy(rows2_v, acc_sh.at[idst2_v], ssem2, add=True)
        s0.wait()
        a2.wait()
        g2 = pltpu.async_copy(t_ref.at[isrc3_v], rows_v, sem)
        s1.wait()
        a3.wait()
        g3 = pltpu.async_copy(t_ref.at[isrc4_v], rows2_v, sem2)
        g2.wait()
        b2.wait()
        s2 = pltpu.async_copy(rows_v, acc_sh.at[idst3_v], ssem, add=True)
        g3.wait()
        b3.wait()
        s3 = pltpu.async_copy(rows2_v, acc_sh.at[idst4_v], ssem2, add=True)
        s2.wait()
        s3.wait()
        return 0
    lax.fori_loop(0, CHUNKS // 4, body, 0)

    plsc.subcore_barrier()
    pltpu.sync_copy(acc_sh.at[pl.ds(s * RPT, RPT)],
                    s_out_ref.at[pl.ds(c * NPAD + s * RPT, RPT)])


def _dense_layer_body(s_ref, c_ref, z_ref,
                      wnu_ref, wsu_ref, bu_ref, wni_ref, wsi_ref, bi_ref,
                      h_ref):
    cu = jnp.clip(c_ref[1, 0, 0, :], 1.0, None)
    ci = jnp.clip(c_ref[0, 0, 0, :], 1.0, None)
    agg_u = s_ref[1] / cu[:, None]
    agg_i = s_ref[0] / ci[:, None]
    nu = (jnp.dot(agg_u, wnu_ref[...], preferred_element_type=jnp.float32)
          + jnp.dot(z_ref[0], wsu_ref[...], preferred_element_type=jnp.float32)
          + bu_ref[0, :])
    ni = (jnp.dot(agg_i, wni_ref[...], preferred_element_type=jnp.float32)
          + jnp.dot(z_ref[1], wsi_ref[...], preferred_element_type=jnp.float32)
          + bi_ref[0, :])
    h_ref[0] = jnp.maximum(nu, 0.0)
    h_ref[1] = jnp.maximum(ni, 0.0)


def _dense_final_body(s_ref, c_ref, z_ref,
                      wnu_ref, wsu_ref, bu_ref, wni_ref, wsi_ref, bi_ref,
                      wp_ref, bp_ref,
                      out_ref):
    cu = jnp.clip(c_ref[1, 0, 0, :], 1.0, None)
    ci = jnp.clip(c_ref[0, 0, 0, :], 1.0, None)
    agg_u = s_ref[1] / cu[:, None]
    agg_i = s_ref[0] / ci[:, None]
    nu = (jnp.dot(agg_u, wnu_ref[...], preferred_element_type=jnp.float32)
          + jnp.dot(z_ref[0], wsu_ref[...], preferred_element_type=jnp.float32)
          + bu_ref[0, :])
    ni = (jnp.dot(agg_i, wni_ref[...], preferred_element_type=jnp.float32)
          + jnp.dot(z_ref[1], wsi_ref[...], preferred_element_type=jnp.float32)
          + bi_ref[0, :])
    hu = jnp.maximum(nu, 0.0)
    hi = jnp.maximum(ni, 0.0)
    res = (jnp.sum(hu * wp_ref[0, :][None, :], axis=1)
           + jnp.sum(hi * wp_ref[1, :][None, :], axis=1)
           + bp_ref[0, 0])
    out_ref[0, 0, :] = res


def _sc_mesh():
    return plsc.VectorSubcoreMesh(core_axis_name="c", subcore_axis_name="s")


def _embed_hist(ids_all, user_table, item_table, dst_all):
    return pl.kernel(
        _embed_hist_body,
        out_type=(
            jax.ShapeDtypeStruct((2 * NPAD, EMB), jnp.float32),
            jax.ShapeDtypeStruct((2 * NPAD,), jnp.float32),
        ),
        mesh=_sc_mesh(),
        scratch_types=[
            pltpu.VMEM((K,), jnp.int32),
            pltpu.VMEM((K,), jnp.int32),
            pltpu.VMEM((K,), jnp.int32),
            pltpu.VMEM((K,), jnp.int32),
            pltpu.VMEM((GK,), jnp.int32),
            pltpu.VMEM((GK, EMB), jnp.float32),
            pltpu.VMEM((K,), jnp.float32),
            pltpu.VMEM((RPT,), jnp.float32),
            pltpu.VMEM_SHARED((NPAD,), jnp.float32),
            pltpu.SemaphoreType.DMA,
            pltpu.SemaphoreType.DMA,
            pltpu.SemaphoreType.DMA,
            pltpu.SemaphoreType.DMA,
            pltpu.SemaphoreType.DMA,
        ],
    )(ids_all, user_table, item_table, dst_all)


def _segsum(table_flat, src_all, dst_all):
    return pl.kernel(
        _segsum_body,
        out_type=jax.ShapeDtypeStruct((2 * NPAD, EMB), jnp.float32),
        mesh=_sc_mesh(),
        scratch_types=[
            pltpu.VMEM((K,), jnp.int32),
            pltpu.VMEM((K,), jnp.int32),
            pltpu.VMEM((K, EMB), jnp.float32),
            pltpu.VMEM((K,), jnp.int32),
            pltpu.VMEM((K,), jnp.int32),
            pltpu.VMEM((K, EMB), jnp.float32),
            pltpu.VMEM((K,), jnp.int32),
            pltpu.VMEM((K,), jnp.int32),
            pltpu.VMEM((K,), jnp.int32),
            pltpu.VMEM((K,), jnp.int32),
            pltpu.VMEM_SHARED((NPAD, EMB), jnp.float32),
            pltpu.SemaphoreType.DMA,
            pltpu.SemaphoreType.DMA,
            pltpu.SemaphoreType.DMA,
            pltpu.SemaphoreType.DMA,
            pltpu.SemaphoreType.DMA,
            pltpu.SemaphoreType.DMA,
            pltpu.SemaphoreType.DMA,
            pltpu.SemaphoreType.DMA,
        ],
    )(table_flat, src_all, dst_all)


def _dense_layer(S, C4, Z, Wnu, Wsu, bu, Wni, Wsi, bi):
    wspec = pl.BlockSpec((EMB, HID), lambda i: (0, 0))
    bspec = pl.BlockSpec((1, HID), lambda i: (0, 0))
    return pl.pallas_call(
        _dense_layer_body,
        grid=(NBLK,),
        in_specs=[
            pl.BlockSpec((2, BLK, EMB), lambda i: (0, i, 0)),
            pl.BlockSpec((2, 1, 1, BLK), lambda i: (0, i, 0, 0)),
            pl.BlockSpec((2, BLK, EMB), lambda i: (0, i, 0)),
            wspec, wspec, bspec, wspec, wspec, bspec,
        ],
        out_specs=pl.BlockSpec((2, BLK, HID), lambda i: (0, i, 0)),
        out_shape=jax.ShapeDtypeStruct((2, NPAD, HID), jnp.float32),
    )(S, C4, Z, Wnu, Wsu, bu, Wni, Wsi, bi)


def _dense_final(S, C4, Z, Wnu, Wsu, bu, Wni, Wsi, bi, Wp2, bp2):
    wspec = pl.BlockSpec((HID, HID), lambda i: (0, 0))
    bspec = pl.BlockSpec((1, HID), lambda i: (0, 0))
    return pl.pallas_call(
        _dense_final_body,
        grid=(NBLK,),
        in_specs=[
            pl.BlockSpec((2, BLK, HID), lambda i: (0, i, 0)),
            pl.BlockSpec((2, 1, 1, BLK), lambda i: (0, i, 0, 0)),
            pl.BlockSpec((2, BLK, HID), lambda i: (0, i, 0)),
            wspec, wspec, bspec, wspec, wspec, bspec,
            pl.BlockSpec((2, HID), lambda i: (0, 0)),
            bspec,
        ],
        out_specs=pl.BlockSpec((1, 1, BLK), lambda i: (i, 0, 0)),
        out_shape=jax.ShapeDtypeStruct((NBLK, 1, BLK), jnp.float32),
    )(S, C4, Z, Wnu, Wsu, bu, Wni, Wsi, bi, Wp2, bp2)


def kernel(user_ids, item_ids, edge_index_u2i, edge_index_i2u,
           user_table, item_table,
           Wn1_u2i, Ws1_u2i, b1_u2i, Wn1_i2u, Ws1_i2u, b1_i2u,
           Wn2_u2i, Ws2_u2i, b2_u2i, Wn2_i2u, Ws2_i2u, b2_i2u,
           Wp, bp):
    idpad = jnp.zeros((NPAD - NU,), jnp.int32)
    ids_all = jnp.concatenate([user_ids, idpad, item_ids, idpad])
    epad = EPAD - E
    srcfill = jnp.arange(epad, dtype=jnp.int32) % NPAD
    dstfill = NU + (jnp.arange(epad, dtype=jnp.int32) % (NPAD - NU))
    src_all = jnp.concatenate([
        edge_index_u2i[0], srcfill,
        edge_index_i2u[0] + NPAD, srcfill + NPAD,
    ])
    dst_all = jnp.concatenate([
        edge_index_u2i[1], dstfill,
        edge_index_i2u[1], dstfill,
    ])

    Zf, counts = _embed_hist(ids_all, user_table, item_table, dst_all)
    Z = Zf.reshape(2, NPAD, EMB)
    C4 = counts.reshape(2, NBLK, 1, BLK)

    b1u = b1_i2u.reshape(1, HID)
    b1i = b1_u2i.reshape(1, HID)
    b2u = b2_i2u.reshape(1, HID)
    b2i = b2_u2i.reshape(1, HID)
    Wp2 = Wp.reshape(2, HID)
    bp2 = jnp.broadcast_to(bp.reshape(1, 1), (1, HID))

    S1 = _segsum(Zf, src_all, dst_all).reshape(2, NPAD, EMB)
    H1 = _dense_layer(S1, C4, Z, Wn1_i2u, Ws1_i2u, b1u, Wn1_u2i, Ws1_u2i, b1i)
    S2 = _segsum(H1.reshape(2 * NPAD, HID), src_all, dst_all).reshape(2, NPAD, HID)
    out = _dense_final(S2, C4, H1, Wn2_i2u, Ws2_i2u, b2u,
                       Wn2_u2i, Ws2_u2i, b2i, Wp2, bp2)
    return out.reshape(NPAD, 1)[:NU]
